# Initial kernel scaffold; baseline (speedup 1.0000x reference)
import jax
import jax.numpy as jnp
from jax.experimental import pallas as pl


def kernel(x, edge_index, W1, att_src1, att_dst1, b1, W2, att_src2, att_dst2, b2):
    raise NotImplementedError("write your pallas kernel here")


# speedup vs baseline: 41.5787x; 41.5787x over previous
import functools
import math

import jax
import jax.numpy as jnp
from jax import lax
from jax.experimental import pallas as pl
from jax.experimental.pallas import tpu as pltpu
from jax.experimental.pallas import tpu_sc as plsc

_N = 10000
_E = 320000
_IN = 128
_HID = 16
_HEADS = 8
_OUT = 16
_BLK = 432
_NBLK = 24
_NP = _BLK * _NBLK
_ROW1 = _HEADS * 16 + 16
_ROW2 = 1 * 16 + 16
_B = 80
_F32 = jnp.float32


def _tca_body(x_ref, w_ref, aa_ref, p_ref, ad_ref, g_ref):
    i = pl.program_id(0)
    h = jnp.dot(x_ref[...], w_ref[...], preferred_element_type=_F32)
    aab = jnp.dot(h, aa_ref[...], preferred_element_type=_F32)
    p_ref[:, 0:128] = h
    p_ref[:, 128:144] = aab
    ad_ref[:, 0:8] = aab[:, 8:16]
    ad_ref[:, 8:16] = jnp.zeros((_BLK, 8), _F32)
    m = jnp.max(aab, axis=0, keepdims=True)
    mfull = jnp.concatenate(
        [jnp.broadcast_to(m, (8, 16)), jnp.full((8, 112), -jnp.inf, _F32)],
        axis=1)

    @pl.when(i == 0)
    def _():
        g_ref[...] = jnp.full((8, 128), -jnp.inf, _F32)

    g_ref[...] = jnp.maximum(g_ref[...], mfull)


def _build_tca(interpret=False):
    return pl.pallas_call(
        _tca_body,
        grid=(_NBLK,),
        in_specs=[
            pl.BlockSpec((_BLK, 128), lambda i: (i, 0)),
            pl.BlockSpec((128, 128), lambda i: (0, 0)),
            pl.BlockSpec((128, 16), lambda i: (0, 0)),
        ],
        out_specs=[
            pl.BlockSpec((_BLK, _ROW1), lambda i: (i, 0)),
            pl.BlockSpec((_BLK, 16), lambda i: (i, 0)),
            pl.BlockSpec((8, 128), lambda i: (0, 0)),
        ],
        out_shape=[
            jax.ShapeDtypeStruct((_NP, _ROW1), _F32),
            jax.ShapeDtypeStruct((_NP, 16), _F32),
            jax.ShapeDtypeStruct((8, 128), _F32),
        ],
        interpret=interpret,
    )


def _tcb_body(a_ref, r_ref, b1_ref, w2_ref, a2_ref, p2_ref,
              pk_ref, ad_ref, g_ref):
    i = pl.program_id(0)
    acc = a_ref[0] + a_ref[1]
    num = acc[:, 0:128]
    den8 = acc[:, 128:136]
    den = jnp.dot(den8, r_ref[...], preferred_element_type=_F32)
    out1 = num / (den + 1e-16) + b1_ref[...]
    helu = jnp.where(out1 > 0, out1, jnp.exp(jnp.minimum(out1, 0.0)) - 1.0)
    h2 = jnp.dot(helu, w2_ref[...], preferred_element_type=_F32)
    a2 = jnp.dot(h2, a2_ref[...], preferred_element_type=_F32)
    pk_ref[:, 0:16] = h2
    pk_ref[:, 16:32] = a2
    ad_ref[...] = jnp.dot(a2, p2_ref[...], preferred_element_type=_F32)
    m = jnp.max(a2, axis=0, keepdims=True)
    mfull = jnp.concatenate(
        [jnp.broadcast_to(m, (8, 16)), jnp.full((8, 112), -jnp.inf, _F32)],
        axis=1)

    @pl.when(i == 0)
    def _():
        g_ref[...] = jnp.full((8, 128), -jnp.inf, _F32)

    g_ref[...] = jnp.maximum(g_ref[...], mfull)


def _build_tcb(interpret=False):
    return pl.pallas_call(
        _tcb_body,
        grid=(_NBLK,),
        in_specs=[
            pl.BlockSpec((2, _BLK, _ROW1), lambda i: (0, i, 0)),
            pl.BlockSpec((8, 128), lambda i: (0, 0)),
            pl.BlockSpec((1, 128), lambda i: (0, 0)),
            pl.BlockSpec((128, 16), lambda i: (0, 0)),
            pl.BlockSpec((16, 16), lambda i: (0, 0)),
            pl.BlockSpec((16, 16), lambda i: (0, 0)),
        ],
        out_specs=[
            pl.BlockSpec((_BLK, _ROW2), lambda i: (i, 0)),
            pl.BlockSpec((_BLK, 16), lambda i: (i, 0)),
            pl.BlockSpec((8, 128), lambda i: (0, 0)),
        ],
        out_shape=[
            jax.ShapeDtypeStruct((_NP, _ROW2), _F32),
            jax.ShapeDtypeStruct((_NP, 16), _F32),
            jax.ShapeDtypeStruct((8, 128), _F32),
        ],
        interpret=interpret,
    )


def _tcc_body(a_ref, b2_ref, o_ref):
    acc = a_ref[0] + a_ref[1]
    num = acc[:, 0:16]
    den = acc[:, 16:17]
    o = num / (den + 1e-16) + b2_ref[...]
    m = jnp.max(o, axis=1, keepdims=True)
    sh = o - m
    lse = jnp.log(jnp.sum(jnp.exp(sh), axis=1, keepdims=True))
    o_ref[...] = sh - lse


def _build_tcc(interpret=False):
    return pl.pallas_call(
        _tcc_body,
        grid=(_NBLK,),
        in_specs=[
            pl.BlockSpec((2, _BLK, _ROW2), lambda i: (0, i, 0)),
            pl.BlockSpec((1, 16), lambda i: (0, 0)),
        ],
        out_specs=pl.BlockSpec((_BLK, 16), lambda i: (i, 0)),
        out_shape=jax.ShapeDtypeStruct((_NP, 16), _F32),
        interpret=interpret,
    )


def _build_sc_edge(nheads, nc, ns, ep, interpret=False):
    row = nheads * 16 + 16
    att_off = nheads * 16
    nw = nc * ns
    nchunks = ep // _B
    rows_per_sub = _NP // ns
    mesh = plsc.VectorSubcoreMesh(
        core_axis_name="c", subcore_axis_name="s",
        num_cores=nc, num_subcores=ns)

    @functools.partial(
        pl.kernel,
        out_type=jax.ShapeDtypeStruct((nc, _NP, row), _F32),
        mesh=mesh,
        scratch_types=[
            pltpu.VMEM((_B,), jnp.int32),
            pltpu.VMEM((_B,), jnp.int32),
            pltpu.VMEM((_B, row), _F32),
            pltpu.VMEM((_B, 16), _F32),
            pltpu.VMEM((_B, row), _F32),
            pltpu.VMEM((16,), _F32),
            pltpu.VMEM_SHARED((_NP, row), _F32),
            pltpu.SemaphoreType.DMA,
        ],
        compiler_params=pltpu.CompilerParams(use_tc_tiling_on_sc=False),
        interpret=interpret,
    )
    def sc_edge(pack_hbm, adt_hbm, src_hbm, dst_hbm, gmax_hbm, zrow_hbm,
                out_hbm, idx_s, idx_d, gbuf, abuf, sbuf, gmaxv, acc, sem):
        c = lax.axis_index("c")
        s = lax.axis_index("s")
        w = s * nc + c
        pltpu.sync_copy(zrow_hbm, acc.at[pl.ds(s * rows_per_sub, rows_per_sub)])
        pltpu.sync_copy(gmax_hbm, gmaxv)
        plsc.subcore_barrier()
        gv = gmaxv[...]
        lanemask = lax.iota(jnp.int32, 16) < nheads

        def chunk_body(k2, carry):
            base = w * ep + k2 * _B
            pltpu.sync_copy(src_hbm.at[pl.ds(base, _B)], idx_s)
            pltpu.sync_copy(dst_hbm.at[pl.ds(base, _B)], idx_d)
            pltpu.async_copy(pack_hbm.at[idx_s], gbuf, sem).wait()
            pltpu.async_copy(adt_hbm.at[idx_d], abuf, sem).wait()

            def edge_body(e, carry2):
                va = gbuf[e, pl.ds(att_off, 16)]
                vb = abuf[e, :]
                al = va + vb
                al = jnp.where(al >= 0.0, al, al * 0.2)
                ea = jnp.exp(al - gv)
                ea = jnp.where(lanemask, ea, 0.0)
                sbuf[e, pl.ds(att_off, 16)] = ea
                for j in range(nheads):
                    sj = ea[j]
                    sbuf[e, pl.ds(j * 16, 16)] = gbuf[e, pl.ds(j * 16, 16)] * sj
                return carry2

            lax.fori_loop(0, _B, edge_body, 0)
            pltpu.sync_copy(sbuf, acc.at[idx_d], add=True)
            return carry

        lax.fori_loop(0, nchunks, chunk_body, 0)
        plsc.subcore_barrier()
        pltpu.sync_copy(
            acc.at[pl.ds(s * rows_per_sub, rows_per_sub)],
            out_hbm.at[c, pl.ds(s * rows_per_sub, rows_per_sub)])

    return sc_edge


def _sc_geometry():
    try:
        info = plsc.get_sparse_core_info()
        nc, ns = info.num_cores, info.num_subcores
    except Exception:
        nc, ns = 2, 16
    return nc, ns


def kernel(x, edge_index, W1, att_src1, att_dst1, b1, W2, att_src2,
           att_dst2, b2):
    nc, ns = _sc_geometry()
    nw = nc * ns
    etot = _E + _N
    ep = math.ceil(etot / (nw * _B)) * _B
    npad = nw * ep - etot
    loop = jnp.arange(_N, dtype=jnp.int32)
    padidx = _N + (jnp.arange(npad, dtype=jnp.int32) % 16)
    src = jnp.concatenate([edge_index[0].astype(jnp.int32), loop, padidx])
    dst = jnp.concatenate([edge_index[1].astype(jnp.int32), loop, padidx])

    xp = jnp.pad(x, ((0, _NP - _N), (0, 0)))
    rows = jnp.arange(128)
    hcol = jnp.repeat(jnp.arange(8), 16)
    AA = jnp.zeros((128, 16), _F32)
    AA = AA.at[rows, hcol].set(att_src1.reshape(-1))
    AA = AA.at[rows, hcol + 8].set(att_dst1.reshape(-1))

    tca = _build_tca()
    pack1, adt1, gmacc = tca(xp, W1, AA)
    gm8 = gmacc[0, 0:8] + gmacc[0, 8:16]
    gmax16 = jnp.concatenate([gm8, gm8])

    sc1 = _build_sc_edge(_HEADS, nc, ns, ep)
    z1 = jnp.zeros((_NP // ns, _ROW1), _F32)
    acc1 = sc1(pack1, adt1, src, dst, gmax16, z1)

    R = (jnp.arange(128)[None, :] // 16 == jnp.arange(8)[:, None]).astype(_F32)
    A2 = jnp.zeros((16, 16), _F32)
    A2 = A2.at[:, 0].set(att_src2[0]).at[:, 1].set(att_dst2[0])
    P2 = jnp.zeros((16, 16), _F32).at[1, 0].set(1.0)
    tcb = _build_tcb()
    pack2, adt2, gm2acc = tcb(acc1, R, b1.reshape(1, 128), W2, A2, P2)
    g2 = gm2acc[0, 0] + gm2acc[0, 1]
    gmax2 = jnp.full((16,), g2, _F32)

    sc2 = _build_sc_edge(1, nc, ns, ep)
    z2 = jnp.zeros((_NP // ns, _ROW2), _F32)
    acc2 = sc2(pack2, adt2, src, dst, gmax2, z2)

    tcc = _build_tcc()
    o = tcc(acc2, b2.reshape(1, 16))
    return o[:_N]


# speedup vs baseline: 43.3169x; 1.0418x over previous
import functools
import math

import jax
import jax.numpy as jnp
from jax import lax
from jax.experimental import pallas as pl
from jax.experimental.pallas import tpu as pltpu
from jax.experimental.pallas import tpu_sc as plsc

_N = 10000
_E = 320000
_IN = 128
_HID = 16
_HEADS = 8
_OUT = 16
_BLK = 632
_NBLK = 16
_NP = _BLK * _NBLK
_ROW1 = _HEADS * 16 + 16
_ROW2 = 1 * 16 + 16
_F32 = jnp.float32


def _tca_body(x_ref, w_ref, aa_ref, p_ref, ad_ref, g_ref):
    i = pl.program_id(0)
    h = jnp.dot(x_ref[...], w_ref[...], preferred_element_type=_F32)
    aab = jnp.dot(h, aa_ref[...], preferred_element_type=_F32)
    p_ref[:, 0:128] = h
    p_ref[:, 128:144] = aab
    ad_ref[:, 0:8] = aab[:, 8:16]
    ad_ref[:, 8:16] = jnp.zeros((_BLK, 8), _F32)
    m = jnp.max(aab, axis=0, keepdims=True)
    mfull = jnp.concatenate(
        [jnp.broadcast_to(m, (8, 16)), jnp.full((8, 112), -jnp.inf, _F32)],
        axis=1)

    @pl.when(i == 0)
    def _():
        g_ref[...] = jnp.full((8, 128), -jnp.inf, _F32)

    g_ref[...] = jnp.maximum(g_ref[...], mfull)


def _build_tca(interpret=False):
    return pl.pallas_call(
        _tca_body,
        grid=(_NBLK,),
        in_specs=[
            pl.BlockSpec((_BLK, 128), lambda i: (i, 0)),
            pl.BlockSpec((128, 128), lambda i: (0, 0)),
            pl.BlockSpec((128, 16), lambda i: (0, 0)),
        ],
        out_specs=[
            pl.BlockSpec((_BLK, _ROW1), lambda i: (i, 0)),
            pl.BlockSpec((_BLK, 16), lambda i: (i, 0)),
            pl.BlockSpec((8, 128), lambda i: (0, 0)),
        ],
        out_shape=[
            jax.ShapeDtypeStruct((_NP, _ROW1), _F32),
            jax.ShapeDtypeStruct((_NP, 16), _F32),
            jax.ShapeDtypeStruct((8, 128), _F32),
        ],
        interpret=interpret,
    )


def _tcb_body(a_ref, r_ref, b1_ref, w2_ref, a2_ref, p2_ref,
              pk_ref, ad_ref, g_ref):
    i = pl.program_id(0)
    acc = a_ref[0] + a_ref[1]
    num = acc[:, 0:128]
    den8 = acc[:, 128:136]
    den = jnp.dot(den8, r_ref[...], preferred_element_type=_F32)
    out1 = num / (den + 1e-16) + b1_ref[...]
    helu = jnp.where(out1 > 0, out1, jnp.exp(jnp.minimum(out1, 0.0)) - 1.0)
    h2 = jnp.dot(helu, w2_ref[...], preferred_element_type=_F32)
    a2 = jnp.dot(h2, a2_ref[...], preferred_element_type=_F32)
    pk_ref[:, 0:16] = h2
    pk_ref[:, 16:32] = a2
    ad_ref[...] = jnp.dot(a2, p2_ref[...], preferred_element_type=_F32)
    m = jnp.max(a2, axis=0, keepdims=True)
    mfull = jnp.concatenate(
        [jnp.broadcast_to(m, (8, 16)), jnp.full((8, 112), -jnp.inf, _F32)],
        axis=1)

    @pl.when(i == 0)
    def _():
        g_ref[...] = jnp.full((8, 128), -jnp.inf, _F32)

    g_ref[...] = jnp.maximum(g_ref[...], mfull)


def _build_tcb(interpret=False):
    return pl.pallas_call(
        _tcb_body,
        grid=(_NBLK,),
        in_specs=[
            pl.BlockSpec((2, _BLK, _ROW1), lambda i: (0, i, 0)),
            pl.BlockSpec((8, 128), lambda i: (0, 0)),
            pl.BlockSpec((1, 128), lambda i: (0, 0)),
            pl.BlockSpec((128, 16), lambda i: (0, 0)),
            pl.BlockSpec((16, 16), lambda i: (0, 0)),
            pl.BlockSpec((16, 16), lambda i: (0, 0)),
        ],
        out_specs=[
            pl.BlockSpec((_BLK, _ROW2), lambda i: (i, 0)),
            pl.BlockSpec((_BLK, 16), lambda i: (i, 0)),
            pl.BlockSpec((8, 128), lambda i: (0, 0)),
        ],
        out_shape=[
            jax.ShapeDtypeStruct((_NP, _ROW2), _F32),
            jax.ShapeDtypeStruct((_NP, 16), _F32),
            jax.ShapeDtypeStruct((8, 128), _F32),
        ],
        interpret=interpret,
    )


def _tcc_body(a_ref, b2_ref, o_ref):
    acc = a_ref[0] + a_ref[1]
    num = acc[:, 0:16]
    den = acc[:, 16:17]
    o = num / (den + 1e-16) + b2_ref[...]
    m = jnp.max(o, axis=1, keepdims=True)
    sh = o - m
    lse = jnp.log(jnp.sum(jnp.exp(sh), axis=1, keepdims=True))
    o_ref[...] = sh - lse


def _build_tcc(interpret=False):
    return pl.pallas_call(
        _tcc_body,
        grid=(_NBLK,),
        in_specs=[
            pl.BlockSpec((2, _BLK, _ROW2), lambda i: (0, i, 0)),
            pl.BlockSpec((1, 16), lambda i: (0, 0)),
        ],
        out_specs=pl.BlockSpec((_BLK, 16), lambda i: (i, 0)),
        out_shape=jax.ShapeDtypeStruct((_NP, 16), _F32),
        interpret=interpret,
    )


def _build_sc_edge(nheads, nc, ns, ep, bchunk, interpret=False):
    row = nheads * 16 + 16
    att_off = nheads * 16
    nw = nc * ns
    _B = bchunk
    nchunks = ep // _B
    rows_per_sub = _NP // ns
    mesh = plsc.VectorSubcoreMesh(
        core_axis_name="c", subcore_axis_name="s",
        num_cores=nc, num_subcores=ns)

    @functools.partial(
        pl.kernel,
        out_type=jax.ShapeDtypeStruct((nc, _NP, row), _F32),
        mesh=mesh,
        scratch_types=[
            pltpu.VMEM((_B,), jnp.int32),
            pltpu.VMEM((_B,), jnp.int32),
            pltpu.VMEM((_B, row), _F32),
            pltpu.VMEM((_B, 16), _F32),
            pltpu.VMEM((_B, row), _F32),
            pltpu.VMEM((_B,), jnp.int32),
            pltpu.VMEM((_B,), jnp.int32),
            pltpu.VMEM((_B, row), _F32),
            pltpu.VMEM((_B, 16), _F32),
            pltpu.VMEM((_B, row), _F32),
            pltpu.VMEM((16,), _F32),
            pltpu.VMEM_SHARED((_NP, row), _F32),
            pltpu.SemaphoreType.DMA,
            pltpu.SemaphoreType.DMA,
        ],
        compiler_params=pltpu.CompilerParams(use_tc_tiling_on_sc=False),
        interpret=interpret,
    )
    def sc_edge(pack_hbm, adt_hbm, src_hbm, dst_hbm, gmax_hbm, zrow_hbm,
                out_hbm, is0, id0, g0, a0, s0, is1, id1, g1, a1, s1,
                gmaxv, acc, sem0, sem1):
        c = lax.axis_index("c")
        s = lax.axis_index("s")
        w = s * nc + c
        ebase = w * ep
        pltpu.sync_copy(zrow_hbm, acc.at[pl.ds(s * rows_per_sub, rows_per_sub)])
        pltpu.sync_copy(gmax_hbm, gmaxv)
        plsc.subcore_barrier()
        gv = gmaxv[...]
        lanemask = lax.iota(jnp.int32, 16) < nheads
        bufs = ((is0, id0, g0, a0, s0, sem0), (is1, id1, g1, a1, s1, sem1))

        def load_chunk(k2, b):
            isx, idx, gb, ab, _, sem = bufs[b]
            base = ebase + k2 * _B
            pltpu.sync_copy(src_hbm.at[pl.ds(base, _B)], isx)
            pltpu.sync_copy(dst_hbm.at[pl.ds(base, _B)], idx)
            pltpu.async_copy(pack_hbm.at[isx], gb, sem)
            pltpu.async_copy(adt_hbm.at[idx], ab, sem)

        def wait_chunk(b):
            isx, idx, gb, ab, _, sem = bufs[b]
            pltpu.make_async_copy(pack_hbm.at[isx], gb, sem).wait()
            pltpu.make_async_copy(adt_hbm.at[idx], ab, sem).wait()

        def compute_chunk(b):
            isx, idx, gb, ab, sb, sem = bufs[b]

            def edge_body(e, carry2):
                va = gb[e, pl.ds(att_off, 16)]
                vb = ab[e, :]
                al = va + vb
                al = jnp.where(al >= 0.0, al, al * 0.2)
                ea = jnp.exp(al - gv)
                ea = jnp.where(lanemask, ea, 0.0)
                sb[e, pl.ds(att_off, 16)] = ea
                for j in range(nheads):
                    sj = ea[j]
                    sb[e, pl.ds(j * 16, 16)] = gb[e, pl.ds(j * 16, 16)] * sj
                return carry2

            lax.fori_loop(0, _B, edge_body, 0, unroll=2)
            pltpu.sync_copy(sb, acc.at[idx], add=True)

        load_chunk(0, 0)

        def outer(i2, carry):
            for b in range(2):
                k2 = i2 * 2 + b
                knext = lax.rem(k2 + 1, nchunks)
                load_chunk(knext, 1 - b)
                wait_chunk(b)
                compute_chunk(b)
            return carry

        lax.fori_loop(0, nchunks // 2, outer, 0)
        wait_chunk(0)
        plsc.subcore_barrier()
        pltpu.sync_copy(
            acc.at[pl.ds(s * rows_per_sub, rows_per_sub)],
            out_hbm.at[c, pl.ds(s * rows_per_sub, rows_per_sub)])

    return sc_edge


def _sc_geometry():
    try:
        info = plsc.get_sparse_core_info()
        nc, ns = info.num_cores, info.num_subcores
    except Exception:
        nc, ns = 2, 16
    return nc, ns


def kernel(x, edge_index, W1, att_src1, att_dst1, b1, W2, att_src2,
           att_dst2, b2):
    nc, ns = _sc_geometry()
    nw = nc * ns
    etot = _E + _N
    b1c, b2c = 64, 128
    ep1 = math.ceil(etot / (nw * 2 * b1c)) * 2 * b1c
    ep2 = math.ceil(etot / (nw * 2 * b2c)) * 2 * b2c
    npad = max(nw * ep1, nw * ep2) - etot
    loop = jnp.arange(_N, dtype=jnp.int32)
    padidx = _N + (jnp.arange(npad, dtype=jnp.int32) % 16)
    src = jnp.concatenate([edge_index[0].astype(jnp.int32), loop, padidx])
    dst = jnp.concatenate([edge_index[1].astype(jnp.int32), loop, padidx])

    xp = jnp.pad(x, ((0, _NP - _N), (0, 0)))
    rows = jnp.arange(128)
    hcol = jnp.repeat(jnp.arange(8), 16)
    AA = jnp.zeros((128, 16), _F32)
    AA = AA.at[rows, hcol].set(att_src1.reshape(-1))
    AA = AA.at[rows, hcol + 8].set(att_dst1.reshape(-1))

    tca = _build_tca()
    pack1, adt1, gmacc = tca(xp, W1, AA)
    gm8 = gmacc[0, 0:8] + gmacc[0, 8:16]
    gmax16 = jnp.concatenate([gm8, gm8])

    sc1 = _build_sc_edge(_HEADS, nc, ns, ep1, b1c)
    z1 = jnp.zeros((_NP // ns, _ROW1), _F32)
    acc1 = sc1(pack1, adt1, src, dst, gmax16, z1)

    R = (jnp.arange(128)[None, :] // 16 == jnp.arange(8)[:, None]).astype(_F32)
    A2 = jnp.zeros((16, 16), _F32)
    A2 = A2.at[:, 0].set(att_src2[0]).at[:, 1].set(att_dst2[0])
    P2 = jnp.zeros((16, 16), _F32).at[1, 0].set(1.0)
    tcb = _build_tcb()
    pack2, adt2, gm2acc = tcb(acc1, R, b1.reshape(1, 128), W2, A2, P2)
    g2 = gm2acc[0, 0] + gm2acc[0, 1]
    gmax2 = jnp.full((16,), g2, _F32)

    sc2 = _build_sc_edge(1, nc, ns, ep2, b2c)
    z2 = jnp.zeros((_NP // ns, _ROW2), _F32)
    acc2 = sc2(pack2, adt2, src, dst, gmax2, z2)

    tcc = _build_tcc()
    o = tcc(acc2, b2.reshape(1, 16))
    return o[:_N]


# speedup vs baseline: 110.4077x; 2.5488x over previous
import functools
import math

import jax
import jax.numpy as jnp
from jax import lax
from jax.experimental import pallas as pl
from jax.experimental.pallas import tpu as pltpu
from jax.experimental.pallas import tpu_sc as plsc

_N = 10000
_E = 320000
_IN = 128
_HID = 16
_HEADS = 8
_OUT = 16
_BLK = 632
_NBLK = 16
_NP = _BLK * _NBLK
_ROW1 = _HEADS * 16 + 16
_ROW2 = 1 * 16 + 16
_F32 = jnp.float32


def _tca_body(x_ref, w_ref, aa_ref, p_ref, ad_ref, g_ref):
    i = pl.program_id(0)
    h = jnp.dot(x_ref[...], w_ref[...], preferred_element_type=_F32)
    aab = jnp.dot(h, aa_ref[...], preferred_element_type=_F32)
    p_ref[:, 0:128] = h
    p_ref[:, 128:144] = aab
    ad_ref[:, 0:8] = aab[:, 8:16]
    ad_ref[:, 8:16] = jnp.zeros((_BLK, 8), _F32)
    m = jnp.max(aab, axis=0, keepdims=True)
    mfull = jnp.concatenate(
        [jnp.broadcast_to(m, (8, 16)), jnp.full((8, 112), -jnp.inf, _F32)],
        axis=1)

    @pl.when(i == 0)
    def _():
        g_ref[...] = jnp.full((8, 128), -jnp.inf, _F32)

    g_ref[...] = jnp.maximum(g_ref[...], mfull)


def _build_tca(interpret=False):
    return pl.pallas_call(
        _tca_body,
        grid=(_NBLK,),
        in_specs=[
            pl.BlockSpec((_BLK, 128), lambda i: (i, 0)),
            pl.BlockSpec((128, 128), lambda i: (0, 0)),
            pl.BlockSpec((128, 16), lambda i: (0, 0)),
        ],
        out_specs=[
            pl.BlockSpec((_BLK, _ROW1), lambda i: (i, 0)),
            pl.BlockSpec((_BLK, 16), lambda i: (i, 0)),
            pl.BlockSpec((8, 128), lambda i: (0, 0)),
        ],
        out_shape=[
            jax.ShapeDtypeStruct((_NP, _ROW1), _F32),
            jax.ShapeDtypeStruct((_NP, 16), _F32),
            jax.ShapeDtypeStruct((8, 128), _F32),
        ],
        interpret=interpret,
    )


def _tcb_body(a_ref, r_ref, b1_ref, w2_ref, a2_ref, p2_ref,
              pk_ref, ad_ref, g_ref):
    i = pl.program_id(0)
    acc = a_ref[0] + a_ref[1]
    num = acc[:, 0:128]
    den8 = acc[:, 128:136]
    den = jnp.dot(den8, r_ref[...], preferred_element_type=_F32)
    out1 = num / (den + 1e-16) + b1_ref[...]
    helu = jnp.where(out1 > 0, out1, jnp.exp(jnp.minimum(out1, 0.0)) - 1.0)
    h2 = jnp.dot(helu, w2_ref[...], preferred_element_type=_F32)
    a2 = jnp.dot(h2, a2_ref[...], preferred_element_type=_F32)
    pk_ref[:, 0:16] = h2
    pk_ref[:, 16:32] = a2
    ad_ref[...] = jnp.dot(a2, p2_ref[...], preferred_element_type=_F32)
    m = jnp.max(a2, axis=0, keepdims=True)
    mfull = jnp.concatenate(
        [jnp.broadcast_to(m, (8, 16)), jnp.full((8, 112), -jnp.inf, _F32)],
        axis=1)

    @pl.when(i == 0)
    def _():
        g_ref[...] = jnp.full((8, 128), -jnp.inf, _F32)

    g_ref[...] = jnp.maximum(g_ref[...], mfull)


def _build_tcb(interpret=False):
    return pl.pallas_call(
        _tcb_body,
        grid=(_NBLK,),
        in_specs=[
            pl.BlockSpec((2, _BLK, _ROW1), lambda i: (0, i, 0)),
            pl.BlockSpec((8, 128), lambda i: (0, 0)),
            pl.BlockSpec((1, 128), lambda i: (0, 0)),
            pl.BlockSpec((128, 16), lambda i: (0, 0)),
            pl.BlockSpec((16, 16), lambda i: (0, 0)),
            pl.BlockSpec((16, 16), lambda i: (0, 0)),
        ],
        out_specs=[
            pl.BlockSpec((_BLK, _ROW2), lambda i: (i, 0)),
            pl.BlockSpec((_BLK, 16), lambda i: (i, 0)),
            pl.BlockSpec((8, 128), lambda i: (0, 0)),
        ],
        out_shape=[
            jax.ShapeDtypeStruct((_NP, _ROW2), _F32),
            jax.ShapeDtypeStruct((_NP, 16), _F32),
            jax.ShapeDtypeStruct((8, 128), _F32),
        ],
        interpret=interpret,
    )


def _tcc_body(a_ref, b2_ref, o_ref):
    acc = a_ref[0] + a_ref[1]
    num = acc[:, 0:16]
    den = acc[:, 16:17]
    o = num / (den + 1e-16) + b2_ref[...]
    m = jnp.max(o, axis=1, keepdims=True)
    sh = o - m
    lse = jnp.log(jnp.sum(jnp.exp(sh), axis=1, keepdims=True))
    o_ref[...] = sh - lse


def _build_tcc(interpret=False):
    return pl.pallas_call(
        _tcc_body,
        grid=(_NBLK,),
        in_specs=[
            pl.BlockSpec((2, _BLK, _ROW2), lambda i: (0, i, 0)),
            pl.BlockSpec((1, 16), lambda i: (0, 0)),
        ],
        out_specs=pl.BlockSpec((_BLK, 16), lambda i: (i, 0)),
        out_shape=jax.ShapeDtypeStruct((_NP, 16), _F32),
        interpret=interpret,
    )


def _build_sc_edge(nheads, nc, ns, ep, B, cps, interpret=False):
    row = nheads * 16 + 16
    att_off = nheads * 16
    SB = B * cps
    nsb = ep // SB
    rows_per_sub = _NP // ns
    nvi = B // 16
    mesh = plsc.VectorSubcoreMesh(
        core_axis_name="c", subcore_axis_name="s",
        num_cores=nc, num_subcores=ns)

    @functools.partial(
        pl.kernel,
        out_type=jax.ShapeDtypeStruct((nc, _NP, row), _F32),
        mesh=mesh,
        scratch_types=[
            pltpu.VMEM((SB,), jnp.int32),
            pltpu.VMEM((SB,), jnp.int32),
            pltpu.VMEM((SB,), jnp.int32),
            pltpu.VMEM((SB,), jnp.int32),
            pltpu.VMEM((B,), jnp.int32),
            pltpu.VMEM((B,), jnp.int32),
            pltpu.VMEM((B,), jnp.int32),
            pltpu.VMEM((B,), jnp.int32),
            pltpu.VMEM((B, row), _F32),
            pltpu.VMEM((B, 16), _F32),
            pltpu.VMEM((B, row), _F32),
            pltpu.VMEM((B, 16), _F32),
            pltpu.VMEM((16,), _F32),
            pltpu.VMEM_SHARED((_NP, row), _F32),
            pltpu.SemaphoreType.DMA,
            pltpu.SemaphoreType.DMA,
            pltpu.SemaphoreType.DMA,
            pltpu.SemaphoreType.DMA,
        ],
        compiler_params=pltpu.CompilerParams(
            use_tc_tiling_on_sc=False, needs_layout_passes=False),
        interpret=interpret,
    )
    def sc_edge(pack_hbm, adt_hbm, src_hbm, dst_hbm, gmax_hbm, zrow_hbm,
                out_hbm, ix0, ix1, dx0, dx1, ids0, ids1, idd0, idd1,
                g0, a0, g1, a1, gmaxv, acc, semi0, semi1, semg0, semg1):
        c = lax.axis_index("c")
        s = lax.axis_index("s")
        w = s * nc + c
        ebase = w * ep
        pltpu.sync_copy(zrow_hbm, acc.at[pl.ds(s * rows_per_sub, rows_per_sub)])
        pltpu.sync_copy(gmax_hbm, gmaxv)
        plsc.subcore_barrier()
        gv = gmaxv[...]
        lane = lax.iota(jnp.int32, 16)
        ibufs = ((ix0, dx0, semi0), (ix1, dx1, semi1))
        slots = ((ids0, idd0, g0, a0, semg0), (ids1, idd1, g1, a1, semg1))

        def idx_issue(sidx, ib):
            ix, dx, sem = ibufs[ib]
            base = ebase + sidx * SB
            pltpu.async_copy(src_hbm.at[pl.ds(base, SB)], ix, sem)
            pltpu.async_copy(dst_hbm.at[pl.ds(base, SB)], dx, sem)

        def idx_wait(sidx, ib):
            ix, dx, sem = ibufs[ib]
            base = ebase + sidx * SB
            pltpu.make_async_copy(src_hbm.at[pl.ds(base, SB)], ix, sem).wait()
            pltpu.make_async_copy(dst_hbm.at[pl.ds(base, SB)], dx, sem).wait()

        def gather_issue(ib, j, sl):
            ix, dx, _ = ibufs[ib]
            ids, idd, gb, ab, sem = slots[sl]
            for q in range(nvi):
                ids[pl.ds(q * 16, 16)] = ix[pl.ds(j * B + q * 16, 16)]
                idd[pl.ds(q * 16, 16)] = dx[pl.ds(j * B + q * 16, 16)]
            pltpu.async_copy(pack_hbm.at[ids], gb, sem)
            pltpu.async_copy(adt_hbm.at[idd], ab, sem)

        def gather_wait(sl):
            ids, idd, gb, ab, sem = slots[sl]
            pltpu.make_async_copy(pack_hbm.at[ids], gb, sem).wait()
            pltpu.make_async_copy(adt_hbm.at[idd], ab, sem).wait()

        c_att = jnp.full((16,), att_off, jnp.int32)
        c_zero = jnp.zeros((16,), jnp.int32)

        def compute_chunk(sl):
            ids, idd, gb, ab, sem = slots[sl]
            if nheads == 1:
                def grp(p, carry):
                    e0 = p * 16
                    rws = e0 + lane
                    va = plsc.load_gather(gb, [rws, c_att])
                    vb = plsc.load_gather(ab, [rws, c_zero])
                    al = va + vb
                    al = jnp.where(al >= 0.0, al, al * 0.2)
                    ea = jnp.exp(al - gv)
                    plsc.store_scatter(gb, [rws, c_att], ea)
                    for t in range(16):
                        e = e0 + t
                        gb[e, pl.ds(0, 16)] = gb[e, pl.ds(0, 16)] * ea[t]
                    return carry

                lax.fori_loop(0, B // 16, grp, 0)
            else:
                rofs = lax.shift_right_logical(lane, 3)
                hofs = lane & 7

                def pair(p, carry):
                    e0 = p * 2
                    rws = e0 + rofs
                    cls = att_off + hofs
                    va = plsc.load_gather(gb, [rws, cls])
                    vb = plsc.load_gather(ab, [rws, hofs])
                    al = va + vb
                    al = jnp.where(al >= 0.0, al, al * 0.2)
                    ea = jnp.exp(al - gv)
                    plsc.store_scatter(gb, [rws, cls], ea)
                    for t in range(2):
                        e = e0 + t
                        for j in range(nheads):
                            sj = ea[8 * t + j]
                            gb[e, pl.ds(16 * j, 16)] = (
                                gb[e, pl.ds(16 * j, 16)] * sj)
                    return carry

                lax.fori_loop(0, B // 2, pair, 0)
            pltpu.sync_copy(gb, acc.at[idd], add=True)

        idx_issue(0, 0)
        idx_wait(0, 0)
        gather_issue(0, 0, 0)

        def outer(s2, carry):
            for sb in range(2):
                sidx = s2 * 2 + sb
                nsidx = lax.rem(sidx + 1, nsb)
                idx_issue(nsidx, 1 - sb)
                for j in range(cps):
                    g = sb * cps + j
                    sl = g % 2
                    nsl = (g + 1) % 2
                    if j == cps - 1:
                        idx_wait(nsidx, 1 - sb)
                        gather_issue(1 - sb, 0, nsl)
                    else:
                        gather_issue(sb, j + 1, nsl)
                    gather_wait(sl)
                    compute_chunk(sl)
            return carry

        lax.fori_loop(0, nsb // 2, outer, 0)
        gather_wait(0)
        plsc.subcore_barrier()
        pltpu.sync_copy(
            acc.at[pl.ds(s * rows_per_sub, rows_per_sub)],
            out_hbm.at[c, pl.ds(s * rows_per_sub, rows_per_sub)])

    return sc_edge


def _sc_geometry():
    try:
        info = plsc.get_sparse_core_info()
        nc, ns = info.num_cores, info.num_subcores
    except Exception:
        nc, ns = 2, 16
    return nc, ns


def kernel(x, edge_index, W1, att_src1, att_dst1, b1, W2, att_src2,
           att_dst2, b2):
    nc, ns = _sc_geometry()
    nw = nc * ns
    etot = _E + _N
    bch, cps = 96, 3
    sbsz = bch * cps
    ep1 = math.ceil(etot / (nw * 2 * sbsz)) * 2 * sbsz
    ep2 = ep1
    npad = nw * ep1 - etot
    loop = jnp.arange(_N, dtype=jnp.int32)
    padidx = _N + (jnp.arange(npad, dtype=jnp.int32) % 16)
    src = jnp.concatenate([edge_index[0].astype(jnp.int32), loop, padidx])
    dst = jnp.concatenate([edge_index[1].astype(jnp.int32), loop, padidx])

    xp = jnp.pad(x, ((0, _NP - _N), (0, 0)))
    rows = jnp.arange(128)
    hcol = jnp.repeat(jnp.arange(8), 16)
    AA = jnp.zeros((128, 16), _F32)
    AA = AA.at[rows, hcol].set(att_src1.reshape(-1))
    AA = AA.at[rows, hcol + 8].set(att_dst1.reshape(-1))

    tca = _build_tca()
    pack1, adt1, gmacc = tca(xp, W1, AA)
    gm8 = gmacc[0, 0:8] + gmacc[0, 8:16]
    gmax16 = jnp.concatenate([gm8, gm8])

    sc1 = _build_sc_edge(_HEADS, nc, ns, ep1, bch, cps)
    z1 = jnp.zeros((_NP // ns, _ROW1), _F32)
    acc1 = sc1(pack1, adt1, src, dst, gmax16, z1)

    R = (jnp.arange(128)[None, :] // 16 == jnp.arange(8)[:, None]).astype(_F32)
    A2 = jnp.zeros((16, 16), _F32)
    A2 = A2.at[:, 0].set(att_src2[0]).at[:, 1].set(att_dst2[0])
    P2 = jnp.zeros((16, 16), _F32).at[1, 0].set(1.0)
    tcb = _build_tcb()
    pack2, adt2, gm2acc = tcb(acc1, R, b1.reshape(1, 128), W2, A2, P2)
    g2 = gm2acc[0, 0] + gm2acc[0, 1]
    gmax2 = jnp.full((16,), g2, _F32)

    sc2 = _build_sc_edge(1, nc, ns, ep2, bch, cps)
    z2 = jnp.zeros((_NP // ns, _ROW2), _F32)
    acc2 = sc2(pack2, adt2, src, dst, gmax2, z2)

    tcc = _build_tcc()
    o = tcc(acc2, b2.reshape(1, 16))
    return o[:_N]


# speedup vs baseline: 128.7399x; 1.1660x over previous
import functools
import math

import jax
import jax.numpy as jnp
from jax import lax
from jax.experimental import pallas as pl
from jax.experimental.pallas import tpu as pltpu
from jax.experimental.pallas import tpu_sc as plsc

_N = 10000
_E = 320000
_IN = 128
_HID = 16
_HEADS = 8
_OUT = 16
_BLK = 632
_NBLK = 16
_NP = _BLK * _NBLK
_ROW1 = _HEADS * 16 + 16
_ROW2 = 1 * 16 + 16
_F32 = jnp.float32


def _tca_body(x_ref, w_ref, aa_ref, p_ref, ad_ref, g_ref, scr):
    i = pl.program_id(0)
    h = jnp.dot(x_ref[...], w_ref[...], preferred_element_type=_F32)
    aab = jnp.dot(h, aa_ref[...], preferred_element_type=_F32)
    p_ref[:, 0:128] = h
    p_ref[:, 128:144] = aab
    ad_ref[:, 0:8] = aab[:, 8:16]
    ad_ref[:, 8:16] = jnp.zeros((_BLK, 8), _F32)
    m = jnp.max(aab, axis=0, keepdims=True)
    mfull = jnp.concatenate(
        [jnp.broadcast_to(m, (8, 16)), jnp.full((8, 112), -jnp.inf, _F32)],
        axis=1)

    @pl.when(i == 0)
    def _():
        scr[...] = jnp.full((8, 128), -jnp.inf, _F32)

    scr[...] = jnp.maximum(scr[...], mfull)

    @pl.when(i == _NBLK - 1)
    def _():
        g8 = scr[0:1, 0:8] + scr[0:1, 8:16]
        g_ref[0:1, 0:8] = g8
        g_ref[0:1, 8:16] = g8


def _build_tca(interpret=False):
    return pl.pallas_call(
        _tca_body,
        grid=(_NBLK,),
        in_specs=[
            pl.BlockSpec((_BLK, 128), lambda i: (i, 0)),
            pl.BlockSpec((128, 128), lambda i: (0, 0)),
            pl.BlockSpec((128, 16), lambda i: (0, 0)),
        ],
        out_specs=[
            pl.BlockSpec((_BLK, _ROW1), lambda i: (i, 0)),
            pl.BlockSpec((_BLK, 16), lambda i: (i, 0)),
            pl.BlockSpec((1, 16), lambda i: (0, 0)),
        ],
        out_shape=[
            jax.ShapeDtypeStruct((_NP, _ROW1), _F32),
            jax.ShapeDtypeStruct((_NP, 16), _F32),
            jax.ShapeDtypeStruct((1, 16), _F32),
        ],
        scratch_shapes=[pltpu.VMEM((8, 128), _F32)],
        interpret=interpret,
    )


def _tcb_body(a_ref, r_ref, b1_ref, w2_ref, a2_ref, p2_ref,
              pk_ref, ad_ref, g_ref, scr):
    i = pl.program_id(0)
    acc = a_ref[0] + a_ref[1]
    num = acc[:, 0:128]
    den8 = acc[:, 128:136]
    den = jnp.dot(den8, r_ref[...], preferred_element_type=_F32)
    out1 = num / (den + 1e-16) + b1_ref[...]
    helu = jnp.where(out1 > 0, out1, jnp.exp(jnp.minimum(out1, 0.0)) - 1.0)
    h2 = jnp.dot(helu, w2_ref[...], preferred_element_type=_F32)
    a2 = jnp.dot(h2, a2_ref[...], preferred_element_type=_F32)
    pk_ref[:, 0:16] = h2
    pk_ref[:, 16:32] = a2
    ad_ref[...] = jnp.dot(a2, p2_ref[...], preferred_element_type=_F32)
    m = jnp.max(a2, axis=0, keepdims=True)
    mfull = jnp.concatenate(
        [jnp.broadcast_to(m, (8, 16)), jnp.full((8, 112), -jnp.inf, _F32)],
        axis=1)

    @pl.when(i == 0)
    def _():
        scr[...] = jnp.full((8, 128), -jnp.inf, _F32)

    scr[...] = jnp.maximum(scr[...], mfull)

    @pl.when(i == _NBLK - 1)
    def _():
        g2 = scr[0:1, 0:1] + scr[0:1, 1:2]
        g_ref[...] = jnp.broadcast_to(g2, (1, 16))


def _build_tcb(interpret=False):
    return pl.pallas_call(
        _tcb_body,
        grid=(_NBLK,),
        in_specs=[
            pl.BlockSpec((2, _BLK, _ROW1), lambda i: (0, i, 0)),
            pl.BlockSpec((8, 128), lambda i: (0, 0)),
            pl.BlockSpec((1, 128), lambda i: (0, 0)),
            pl.BlockSpec((128, 16), lambda i: (0, 0)),
            pl.BlockSpec((16, 16), lambda i: (0, 0)),
            pl.BlockSpec((16, 16), lambda i: (0, 0)),
        ],
        out_specs=[
            pl.BlockSpec((_BLK, _ROW2), lambda i: (i, 0)),
            pl.BlockSpec((_BLK, 16), lambda i: (i, 0)),
            pl.BlockSpec((1, 16), lambda i: (0, 0)),
        ],
        out_shape=[
            jax.ShapeDtypeStruct((_NP, _ROW2), _F32),
            jax.ShapeDtypeStruct((_NP, 16), _F32),
            jax.ShapeDtypeStruct((1, 16), _F32),
        ],
        scratch_shapes=[pltpu.VMEM((8, 128), _F32)],
        interpret=interpret,
    )


def _tcc_body(a_ref, b2_ref, o_ref):
    acc = a_ref[0] + a_ref[1]
    num = acc[:, 0:16]
    den = acc[:, 16:17]
    o = num / (den + 1e-16) + b2_ref[...]
    m = jnp.max(o, axis=1, keepdims=True)
    sh = o - m
    lse = jnp.log(jnp.sum(jnp.exp(sh), axis=1, keepdims=True))
    o_ref[...] = sh - lse


def _build_tcc(interpret=False):
    return pl.pallas_call(
        _tcc_body,
        grid=(_NBLK,),
        in_specs=[
            pl.BlockSpec((2, _BLK, _ROW2), lambda i: (0, i, 0)),
            pl.BlockSpec((1, 16), lambda i: (0, 0)),
        ],
        out_specs=pl.BlockSpec((_BLK, 16), lambda i: (i, 0)),
        out_shape=jax.ShapeDtypeStruct((_NP, 16), _F32),
        interpret=interpret,
    )


def _build_sc_edge(nheads, nc, ns, ep, B, cps, interpret=False):
    row = nheads * 16 + 16
    att_off = nheads * 16
    SB = B * cps
    nsb = ep // SB
    rows_per_sub = _NP // ns
    nvi = B // 16
    mesh = plsc.VectorSubcoreMesh(
        core_axis_name="c", subcore_axis_name="s",
        num_cores=nc, num_subcores=ns)

    @functools.partial(
        pl.kernel,
        out_type=jax.ShapeDtypeStruct((nc, _NP, row), _F32),
        mesh=mesh,
        scratch_types=[
            pltpu.VMEM((SB,), jnp.int32),
            pltpu.VMEM((SB,), jnp.int32),
            pltpu.VMEM((SB,), jnp.int32),
            pltpu.VMEM((SB,), jnp.int32),
            pltpu.VMEM((B,), jnp.int32),
            pltpu.VMEM((B,), jnp.int32),
            pltpu.VMEM((B,), jnp.int32),
            pltpu.VMEM((B,), jnp.int32),
            pltpu.VMEM((B,), jnp.int32),
            pltpu.VMEM((B,), jnp.int32),
            pltpu.VMEM((B, row), _F32),
            pltpu.VMEM((B, 16), _F32),
            pltpu.VMEM((B, row), _F32),
            pltpu.VMEM((B, 16), _F32),
            pltpu.VMEM((B, row), _F32),
            pltpu.VMEM((B, 16), _F32),
            pltpu.VMEM((16,), _F32),
            pltpu.VMEM_SHARED((_NP, row), _F32),
            pltpu.SemaphoreType.DMA,
            pltpu.SemaphoreType.DMA,
            pltpu.SemaphoreType.DMA,
            pltpu.SemaphoreType.DMA,
            pltpu.SemaphoreType.DMA,
            pltpu.SemaphoreType.DMA,
            pltpu.SemaphoreType.DMA,
            pltpu.SemaphoreType.DMA,
        ],
        compiler_params=pltpu.CompilerParams(
            use_tc_tiling_on_sc=False, needs_layout_passes=False),
        interpret=interpret,
    )
    def sc_edge(pack_hbm, adt_hbm, src_hbm, dst_hbm, gmax_hbm, zrow_hbm,
                out_hbm, ix0, ix1, dx0, dx1, ids0, ids1, ids2, idd0, idd1,
                idd2, g0, a0, g1, a1, g2, a2, gmaxv, acc, semi0, semi1,
                semg0, semg1, semg2, sems0, sems1, sems2):
        c = lax.axis_index("c")
        s = lax.axis_index("s")
        w = s * nc + c
        ebase = w * ep
        pltpu.sync_copy(zrow_hbm, acc.at[pl.ds(s * rows_per_sub, rows_per_sub)])
        pltpu.sync_copy(gmax_hbm, gmaxv)
        plsc.subcore_barrier()
        gv = gmaxv[...]
        lane = lax.iota(jnp.int32, 16)
        ibufs = ((ix0, dx0, semi0), (ix1, dx1, semi1))
        slots = ((ids0, idd0, g0, a0, semg0, sems0),
                 (ids1, idd1, g1, a1, semg1, sems1),
                 (ids2, idd2, g2, a2, semg2, sems2))

        def idx_issue(sidx, ib):
            ix, dx, sem = ibufs[ib]
            base = ebase + sidx * SB
            pltpu.async_copy(src_hbm.at[pl.ds(base, SB)], ix, sem)
            pltpu.async_copy(dst_hbm.at[pl.ds(base, SB)], dx, sem)

        def idx_wait(sidx, ib):
            ix, dx, sem = ibufs[ib]
            base = ebase + sidx * SB
            pltpu.make_async_copy(src_hbm.at[pl.ds(base, SB)], ix, sem).wait()
            pltpu.make_async_copy(dst_hbm.at[pl.ds(base, SB)], dx, sem).wait()

        def gather_issue(ib, j, sl):
            ix, dx, _ = ibufs[ib]
            ids, idd, gb, ab, sem, _ = slots[sl]
            for q in range(nvi):
                ids[pl.ds(q * 16, 16)] = ix[pl.ds(j * B + q * 16, 16)]
                idd[pl.ds(q * 16, 16)] = dx[pl.ds(j * B + q * 16, 16)]
            pltpu.async_copy(pack_hbm.at[ids], gb, sem)
            pltpu.async_copy(adt_hbm.at[idd], ab, sem)

        def gather_wait(sl):
            ids, idd, gb, ab, sem, _ = slots[sl]
            pltpu.make_async_copy(pack_hbm.at[ids], gb, sem).wait()
            pltpu.make_async_copy(adt_hbm.at[idd], ab, sem).wait()

        def scatter_wait(sl):
            ids, idd, gb, ab, _, sem = slots[sl]
            pltpu.make_async_copy(gb, acc.at[idd], sem).wait()

        c_att = jnp.full((16,), att_off, jnp.int32)
        c_zero = jnp.zeros((16,), jnp.int32)

        def compute_chunk(sl):
            ids, idd, gb, ab, _, sem = slots[sl]
            if nheads == 1:
                def grp(p, carry):
                    e0 = p * 16
                    rws = e0 + lane
                    va = plsc.load_gather(gb, [rws, c_att])
                    vb = plsc.load_gather(ab, [rws, c_zero])
                    al = va + vb
                    al = jnp.where(al >= 0.0, al, al * 0.2)
                    ea = jnp.exp(al - gv)
                    plsc.store_scatter(gb, [rws, c_att], ea)
                    for t in range(16):
                        e = e0 + t
                        gb[e, pl.ds(0, 16)] = gb[e, pl.ds(0, 16)] * ea[t]
                    return carry

                lax.fori_loop(0, B // 16, grp, 0, unroll=2)
            else:
                rofs = lax.shift_right_logical(lane, 3)
                hofs = lane & 7

                def pair(p, carry):
                    e0 = p * 2
                    rws = e0 + rofs
                    cls = att_off + hofs
                    va = plsc.load_gather(gb, [rws, cls])
                    vb = plsc.load_gather(ab, [rws, hofs])
                    al = va + vb
                    al = jnp.where(al >= 0.0, al, al * 0.2)
                    ea = jnp.exp(al - gv)
                    plsc.store_scatter(gb, [rws, cls], ea)
                    for t in range(2):
                        e = e0 + t
                        for j in range(nheads):
                            sj = ea[8 * t + j]
                            gb[e, pl.ds(16 * j, 16)] = (
                                gb[e, pl.ds(16 * j, 16)] * sj)
                    return carry

                lax.fori_loop(0, B // 2, pair, 0, unroll=2)
            pltpu.async_copy(gb, acc.at[idd], sem, add=True)

        idx_issue(0, 0)
        idx_wait(0, 0)
        gather_issue(0, 0, 0)

        def outer(s2, carry):
            for sb in range(2):
                sidx = s2 * 2 + sb
                nsidx = lax.rem(sidx + 1, nsb)
                idx_issue(nsidx, 1 - sb)
                for j in range(cps):
                    sl = j
                    nsl = (j + 1) % 3
                    if sb == 0 and j < 2:
                        @pl.when(s2 > 0)
                        def _():
                            scatter_wait(nsl)
                    else:
                        scatter_wait(nsl)
                    if j == cps - 1:
                        idx_wait(nsidx, 1 - sb)
                        gather_issue(1 - sb, 0, nsl)
                    else:
                        gather_issue(sb, j + 1, nsl)
                    gather_wait(sl)
                    compute_chunk(sl)
            return carry

        lax.fori_loop(0, nsb // 2, outer, 0)
        gather_wait(0)
        scatter_wait(1)
        scatter_wait(2)
        plsc.subcore_barrier()
        pltpu.sync_copy(
            acc.at[pl.ds(s * rows_per_sub, rows_per_sub)],
            out_hbm.at[c, pl.ds(s * rows_per_sub, rows_per_sub)])

    return sc_edge


def _sc_geometry():
    try:
        info = plsc.get_sparse_core_info()
        nc, ns = info.num_cores, info.num_subcores
    except Exception:
        nc, ns = 2, 16
    return nc, ns


def kernel(x, edge_index, W1, att_src1, att_dst1, b1, W2, att_src2,
           att_dst2, b2):
    nc, ns = _sc_geometry()
    nw = nc * ns
    etot = _E + _N
    b1c, b2c, cps = 80, 96, 3
    ep1 = math.ceil(etot / (nw * 2 * b1c * cps)) * 2 * b1c * cps
    ep2 = math.ceil(etot / (nw * 2 * b2c * cps)) * 2 * b2c * cps
    npad = max(nw * ep1, nw * ep2) - etot
    loop = jnp.arange(_N, dtype=jnp.int32)
    padidx = _N + (jnp.arange(npad, dtype=jnp.int32) % 16)
    src = jnp.concatenate([edge_index[0].astype(jnp.int32), loop, padidx])
    dst = jnp.concatenate([edge_index[1].astype(jnp.int32), loop, padidx])

    xp = jnp.pad(x, ((0, _NP - _N), (0, 0)))
    mask8 = (jnp.arange(128)[:, None] // 16 == jnp.arange(8)[None, :])
    AA = jnp.concatenate([
        jnp.where(mask8, att_src1.reshape(-1)[:, None], 0.0),
        jnp.where(mask8, att_dst1.reshape(-1)[:, None], 0.0)], axis=1)

    tca = _build_tca()
    pack1, adt1, gmacc = tca(xp, W1, AA)
    gmax16 = gmacc.reshape(16)

    sc1 = _build_sc_edge(_HEADS, nc, ns, ep1, b1c, cps)
    z1 = jnp.zeros((_NP // ns, _ROW1), _F32)
    acc1 = sc1(pack1, adt1, src, dst, gmax16, z1)

    R = (jnp.arange(128)[None, :] // 16 == jnp.arange(8)[:, None]).astype(_F32)
    A2 = jnp.zeros((16, 16), _F32)
    A2 = A2.at[:, 0].set(att_src2[0]).at[:, 1].set(att_dst2[0])
    P2 = jnp.zeros((16, 16), _F32).at[1, 0].set(1.0)
    tcb = _build_tcb()
    pack2, adt2, gm2acc = tcb(acc1, R, b1.reshape(1, 128), W2, A2, P2)
    gmax2 = gm2acc.reshape(16)

    sc2 = _build_sc_edge(1, nc, ns, ep2, b2c, cps)
    z2 = jnp.zeros((_NP // ns, _ROW2), _F32)
    acc2 = sc2(pack2, adt2, src, dst, gmax2, z2)

    tcc = _build_tcc()
    o = tcc(acc2, b2.reshape(1, 16))
    return o[:_N]


# speedup vs baseline: 135.7075x; 1.0541x over previous
import functools
import math

import jax
import jax.numpy as jnp
from jax import lax
from jax.experimental import pallas as pl
from jax.experimental.pallas import tpu as pltpu
from jax.experimental.pallas import tpu_sc as plsc

_N = 10000
_E = 320000
_IN = 128
_HID = 16
_HEADS = 8
_OUT = 16
_BLK = 632
_NBLK = 16
_NP = _BLK * _NBLK
_ROW1 = _HEADS * 16 + 16
_ROW2 = 1 * 16 + 16
_F32 = jnp.float32


def _tca_body(x_ref, w_ref, aa_ref, p_ref, ad_ref, g_ref, scr):
    i = pl.program_id(0)
    h = jnp.dot(x_ref[...], w_ref[...], preferred_element_type=_F32)
    aab = jnp.dot(h, aa_ref[...], preferred_element_type=_F32)
    p_ref[:, 0:128] = h
    p_ref[:, 128:144] = aab
    ad_ref[:, 0:8] = aab[:, 8:16]
    ad_ref[:, 8:16] = jnp.zeros((_BLK, 8), _F32)
    m = jnp.max(aab, axis=0, keepdims=True)
    mfull = jnp.concatenate(
        [jnp.broadcast_to(m, (8, 16)), jnp.full((8, 112), -jnp.inf, _F32)],
        axis=1)

    @pl.when(i == 0)
    def _():
        scr[...] = jnp.full((8, 128), -jnp.inf, _F32)

    scr[...] = jnp.maximum(scr[...], mfull)

    @pl.when(i == _NBLK - 1)
    def _():
        g8 = scr[0:1, 0:8] + scr[0:1, 8:16]
        g_ref[0:1, 0:8] = g8
        g_ref[0:1, 8:16] = g8


def _build_tca(interpret=False):
    return pl.pallas_call(
        _tca_body,
        grid=(_NBLK,),
        in_specs=[
            pl.BlockSpec((_BLK, 128), lambda i: (i, 0)),
            pl.BlockSpec((128, 128), lambda i: (0, 0)),
            pl.BlockSpec((128, 16), lambda i: (0, 0)),
        ],
        out_specs=[
            pl.BlockSpec((_BLK, _ROW1), lambda i: (i, 0)),
            pl.BlockSpec((_BLK, 16), lambda i: (i, 0)),
            pl.BlockSpec((1, 16), lambda i: (0, 0)),
        ],
        out_shape=[
            jax.ShapeDtypeStruct((_NP, _ROW1), _F32),
            jax.ShapeDtypeStruct((_NP, 16), _F32),
            jax.ShapeDtypeStruct((1, 16), _F32),
        ],
        scratch_shapes=[pltpu.VMEM((8, 128), _F32)],
        interpret=interpret,
    )


def _tcb_body(a_ref, r_ref, b1_ref, w2_ref, a2_ref, p2_ref,
              pk_ref, ad_ref, g_ref, scr):
    i = pl.program_id(0)
    acc = a_ref[0] + a_ref[1]
    num = acc[:, 0:128]
    den8 = acc[:, 128:136]
    den = jnp.dot(den8, r_ref[...], preferred_element_type=_F32)
    out1 = num / (den + 1e-16) + b1_ref[...]
    helu = jnp.where(out1 > 0, out1, jnp.exp(jnp.minimum(out1, 0.0)) - 1.0)
    h2 = jnp.dot(helu, w2_ref[...], preferred_element_type=_F32)
    a2 = jnp.dot(h2, a2_ref[...], preferred_element_type=_F32)
    pk_ref[:, 0:16] = h2
    pk_ref[:, 16:32] = a2
    ad_ref[...] = jnp.dot(a2, p2_ref[...], preferred_element_type=_F32)
    m = jnp.max(a2, axis=0, keepdims=True)
    mfull = jnp.concatenate(
        [jnp.broadcast_to(m, (8, 16)), jnp.full((8, 112), -jnp.inf, _F32)],
        axis=1)

    @pl.when(i == 0)
    def _():
        scr[...] = jnp.full((8, 128), -jnp.inf, _F32)

    scr[...] = jnp.maximum(scr[...], mfull)

    @pl.when(i == _NBLK - 1)
    def _():
        g2 = scr[0:1, 0:1] + scr[0:1, 1:2]
        g_ref[...] = jnp.broadcast_to(g2, (1, 16))


def _build_tcb(interpret=False):
    return pl.pallas_call(
        _tcb_body,
        grid=(_NBLK,),
        in_specs=[
            pl.BlockSpec((2, _BLK, _ROW1), lambda i: (0, i, 0)),
            pl.BlockSpec((8, 128), lambda i: (0, 0)),
            pl.BlockSpec((1, 128), lambda i: (0, 0)),
            pl.BlockSpec((128, 16), lambda i: (0, 0)),
            pl.BlockSpec((16, 16), lambda i: (0, 0)),
            pl.BlockSpec((16, 16), lambda i: (0, 0)),
        ],
        out_specs=[
            pl.BlockSpec((_BLK, _ROW2), lambda i: (i, 0)),
            pl.BlockSpec((_BLK, 16), lambda i: (i, 0)),
            pl.BlockSpec((1, 16), lambda i: (0, 0)),
        ],
        out_shape=[
            jax.ShapeDtypeStruct((_NP, _ROW2), _F32),
            jax.ShapeDtypeStruct((_NP, 16), _F32),
            jax.ShapeDtypeStruct((1, 16), _F32),
        ],
        scratch_shapes=[pltpu.VMEM((8, 128), _F32)],
        interpret=interpret,
    )


def _tcc_body(a_ref, b2_ref, o_ref):
    acc = a_ref[0] + a_ref[1]
    num = acc[:, 0:16]
    den = acc[:, 16:17]
    o = num / (den + 1e-16) + b2_ref[...]
    m = jnp.max(o, axis=1, keepdims=True)
    sh = o - m
    lse = jnp.log(jnp.sum(jnp.exp(sh), axis=1, keepdims=True))
    o_ref[...] = sh - lse


def _build_tcc(interpret=False):
    return pl.pallas_call(
        _tcc_body,
        grid=(_NBLK,),
        in_specs=[
            pl.BlockSpec((2, _BLK, _ROW2), lambda i: (0, i, 0)),
            pl.BlockSpec((1, 16), lambda i: (0, 0)),
        ],
        out_specs=pl.BlockSpec((_BLK, 16), lambda i: (i, 0)),
        out_shape=jax.ShapeDtypeStruct((_NP, 16), _F32),
        interpret=interpret,
    )


def _build_sc_edge(nheads, nc, ns, ep, B, cps, interpret=False):
    row = nheads * 16 + 16
    att_off = nheads * 16
    SB = B * cps
    nsb = ep // SB
    rows_per_sub = _NP // ns
    nvi = B // 16
    mesh = plsc.VectorSubcoreMesh(
        core_axis_name="c", subcore_axis_name="s",
        num_cores=nc, num_subcores=ns)

    @functools.partial(
        pl.kernel,
        out_type=jax.ShapeDtypeStruct((nc, _NP, row), _F32),
        mesh=mesh,
        scratch_types=[
            pltpu.VMEM((SB,), jnp.int32),
            pltpu.VMEM((SB,), jnp.int32),
            pltpu.VMEM((SB,), jnp.int32),
            pltpu.VMEM((SB,), jnp.int32),
            pltpu.VMEM((B,), jnp.int32),
            pltpu.VMEM((B,), jnp.int32),
            pltpu.VMEM((B,), jnp.int32),
            pltpu.VMEM((B,), jnp.int32),
            pltpu.VMEM((B,), jnp.int32),
            pltpu.VMEM((B,), jnp.int32),
            pltpu.VMEM((B, row), _F32),
            pltpu.VMEM((B, 16), _F32),
            pltpu.VMEM((B, row), _F32),
            pltpu.VMEM((B, 16), _F32),
            pltpu.VMEM((B, row), _F32),
            pltpu.VMEM((B, 16), _F32),
            pltpu.VMEM((16,), _F32),
            pltpu.VMEM_SHARED((_NP, row), _F32),
            pltpu.SemaphoreType.DMA,
            pltpu.SemaphoreType.DMA,
            pltpu.SemaphoreType.DMA,
            pltpu.SemaphoreType.DMA,
            pltpu.SemaphoreType.DMA,
            pltpu.SemaphoreType.DMA,
            pltpu.SemaphoreType.DMA,
            pltpu.SemaphoreType.DMA,
        ],
        compiler_params=pltpu.CompilerParams(
            use_tc_tiling_on_sc=False, needs_layout_passes=False),
        interpret=interpret,
    )
    def sc_edge(pack_hbm, adt_hbm, ei_hbm, gmax_hbm, zrow_hbm,
                out_hbm, ix0, ix1, dx0, dx1, ids0, ids1, ids2, idd0, idd1,
                idd2, g0, a0, g1, a1, g2, a2, gmaxv, acc, semi0, semi1,
                semg0, semg1, semg2, sems0, sems1, sems2):
        c = lax.axis_index("c")
        s = lax.axis_index("s")
        w = s * nc + c
        ebase = w * ep
        pltpu.sync_copy(zrow_hbm, acc.at[pl.ds(s * rows_per_sub, rows_per_sub)])
        pltpu.sync_copy(gmax_hbm, gmaxv)
        plsc.subcore_barrier()
        gv = gmaxv[...]
        lane = lax.iota(jnp.int32, 16)
        ibufs = ((ix0, dx0, semi0), (ix1, dx1, semi1))
        slots = ((ids0, idd0, g0, a0, semg0, sems0),
                 (ids1, idd1, g1, a1, semg1, sems1),
                 (ids2, idd2, g2, a2, semg2, sems2))

        def idx_issue(sidx, ib):
            ix, dx, sem = ibufs[ib]
            base = ebase + sidx * SB
            pltpu.async_copy(ei_hbm.at[0, pl.ds(base, SB)], ix, sem)
            pltpu.async_copy(ei_hbm.at[1, pl.ds(base, SB)], dx, sem)

        def idx_wait(sidx, ib):
            ix, dx, sem = ibufs[ib]
            base = ebase + sidx * SB
            pltpu.make_async_copy(ei_hbm.at[0, pl.ds(base, SB)], ix, sem).wait()
            pltpu.make_async_copy(ei_hbm.at[1, pl.ds(base, SB)], dx, sem).wait()

        def gather_issue(ib, j, sl):
            ix, dx, _ = ibufs[ib]
            ids, idd, gb, ab, sem, _ = slots[sl]
            for q in range(nvi):
                ids[pl.ds(q * 16, 16)] = ix[pl.ds(j * B + q * 16, 16)]
                idd[pl.ds(q * 16, 16)] = dx[pl.ds(j * B + q * 16, 16)]
            pltpu.async_copy(pack_hbm.at[ids], gb, sem)
            pltpu.async_copy(adt_hbm.at[idd], ab, sem)

        def gather_wait(sl):
            ids, idd, gb, ab, sem, _ = slots[sl]
            pltpu.make_async_copy(pack_hbm.at[ids], gb, sem).wait()
            pltpu.make_async_copy(adt_hbm.at[idd], ab, sem).wait()

        def scatter_wait(sl):
            ids, idd, gb, ab, _, sem = slots[sl]
            pltpu.make_async_copy(gb, acc.at[idd], sem).wait()

        c_att = jnp.full((16,), att_off, jnp.int32)
        c_zero = jnp.zeros((16,), jnp.int32)

        def compute_chunk(sl):
            ids, idd, gb, ab, _, sem = slots[sl]
            if nheads == 1:
                def grp(p, carry):
                    e0 = p * 16
                    rws = e0 + lane
                    va = plsc.load_gather(gb, [rws, c_att])
                    vb = plsc.load_gather(ab, [rws, c_zero])
                    al = va + vb
                    al = jnp.where(al >= 0.0, al, al * 0.2)
                    ea = jnp.exp(al - gv)
                    plsc.store_scatter(gb, [rws, c_att], ea)
                    for t in range(16):
                        e = e0 + t
                        gb[e, pl.ds(0, 16)] = gb[e, pl.ds(0, 16)] * ea[t]
                    return carry

                lax.fori_loop(0, B // 16, grp, 0, unroll=2)
            else:
                rofs = lax.shift_right_logical(lane, 3)
                hofs = lane & 7

                def pair(p, carry):
                    e0 = p * 2
                    rws = e0 + rofs
                    cls = att_off + hofs
                    va = plsc.load_gather(gb, [rws, cls])
                    vb = plsc.load_gather(ab, [rws, hofs])
                    al = va + vb
                    al = jnp.where(al >= 0.0, al, al * 0.2)
                    ea = jnp.exp(al - gv)
                    plsc.store_scatter(gb, [rws, cls], ea)
                    for t in range(2):
                        e = e0 + t
                        for j in range(nheads):
                            sj = ea[8 * t + j]
                            gb[e, pl.ds(16 * j, 16)] = (
                                gb[e, pl.ds(16 * j, 16)] * sj)
                    return carry

                lax.fori_loop(0, B // 2, pair, 0, unroll=2)
            pltpu.async_copy(gb, acc.at[idd], sem, add=True)

        idx_issue(0, 0)
        idx_wait(0, 0)
        gather_issue(0, 0, 0)

        def outer(s2, carry):
            for sb in range(2):
                sidx = s2 * 2 + sb
                nsidx = lax.rem(sidx + 1, nsb)
                idx_issue(nsidx, 1 - sb)
                for j in range(cps):
                    sl = j
                    nsl = (j + 1) % 3
                    if sb == 0 and j < 2:
                        @pl.when(s2 > 0)
                        def _():
                            scatter_wait(nsl)
                    else:
                        scatter_wait(nsl)
                    if j == cps - 1:
                        idx_wait(nsidx, 1 - sb)
                        gather_issue(1 - sb, 0, nsl)
                    else:
                        gather_issue(sb, j + 1, nsl)
                    gather_wait(sl)
                    compute_chunk(sl)
            return carry

        lax.fori_loop(0, nsb // 2, outer, 0)
        gather_wait(0)
        scatter_wait(1)
        scatter_wait(2)
        plsc.subcore_barrier()
        pltpu.sync_copy(
            acc.at[pl.ds(s * rows_per_sub, rows_per_sub)],
            out_hbm.at[c, pl.ds(s * rows_per_sub, rows_per_sub)])

    return sc_edge


def _sc_geometry():
    try:
        info = plsc.get_sparse_core_info()
        nc, ns = info.num_cores, info.num_subcores
    except Exception:
        nc, ns = 2, 16
    return nc, ns


def kernel(x, edge_index, W1, att_src1, att_dst1, b1, W2, att_src2,
           att_dst2, b2):
    nc, ns = _sc_geometry()
    nw = nc * ns
    etot = _E + _N
    b1c, b2c, cps = 80, 96, 3
    ep1 = math.ceil(etot / (nw * 2 * b1c * cps)) * 2 * b1c * cps
    ep2 = math.ceil(etot / (nw * 2 * b2c * cps)) * 2 * b2c * cps
    npad = max(nw * ep1, nw * ep2) - etot
    loop = jnp.arange(_N, dtype=jnp.int32)
    padidx = _N + (jnp.arange(npad, dtype=jnp.int32) % 16)
    ei = jnp.concatenate(
        [edge_index.astype(jnp.int32),
         jnp.broadcast_to(loop, (2, _N)),
         jnp.broadcast_to(padidx, (2, npad))], axis=1)

    xp = jnp.pad(x, ((0, _NP - _N), (0, 0)))
    mask8 = (jnp.arange(128)[:, None] // 16 == jnp.arange(8)[None, :])
    AA = jnp.concatenate([
        jnp.where(mask8, att_src1.reshape(-1)[:, None], 0.0),
        jnp.where(mask8, att_dst1.reshape(-1)[:, None], 0.0)], axis=1)

    tca = _build_tca()
    pack1, adt1, gmacc = tca(xp, W1, AA)
    gmax16 = gmacc.reshape(16)

    sc1 = _build_sc_edge(_HEADS, nc, ns, ep1, b1c, cps)
    z1 = jnp.zeros((_NP // ns, _ROW1), _F32)
    acc1 = sc1(pack1, adt1, ei, gmax16, z1)

    R = (jnp.arange(128)[None, :] // 16 == jnp.arange(8)[:, None]).astype(_F32)
    A2 = jnp.zeros((16, 16), _F32)
    A2 = A2.at[:, 0].set(att_src2[0]).at[:, 1].set(att_dst2[0])
    P2 = jnp.zeros((16, 16), _F32).at[1, 0].set(1.0)
    tcb = _build_tcb()
    pack2, adt2, gm2acc = tcb(acc1, R, b1.reshape(1, 128), W2, A2, P2)
    gmax2 = gm2acc.reshape(16)

    sc2 = _build_sc_edge(1, nc, ns, ep2, b2c, cps)
    z2 = jnp.zeros((_NP // ns, _ROW2), _F32)
    acc2 = sc2(pack2, adt2, ei, gmax2, z2)

    tcc = _build_tcc()
    o = tcc(acc2, b2.reshape(1, 16))
    return o[:_N]


# speedup vs baseline: 141.1963x; 1.0404x over previous
import functools
import math

import jax
import jax.numpy as jnp
from jax import lax
from jax.experimental import pallas as pl
from jax.experimental.pallas import tpu as pltpu
from jax.experimental.pallas import tpu_sc as plsc

_N = 10000
_E = 320000
_IN = 128
_HID = 16
_HEADS = 8
_OUT = 16
_BLK = 1264
_NBLK = 8
_NP = _BLK * _NBLK
_ROW1 = _HEADS * 16 + 16
_ROW2 = 1 * 16 + 16
_F32 = jnp.float32


def _tca_body(x_ref, w_ref, aa_ref, p_ref, ad_ref, g_ref, scr):
    i = pl.program_id(0)
    h = jnp.dot(x_ref[...], w_ref[...], preferred_element_type=_F32)
    aab = jnp.dot(h, aa_ref[...], preferred_element_type=_F32)
    p_ref[:, 0:128] = h
    p_ref[:, 128:144] = aab
    ad_ref[:, 0:8] = aab[:, 8:16]
    ad_ref[:, 8:16] = jnp.zeros((_BLK, 8), _F32)
    m = jnp.max(aab, axis=0, keepdims=True)
    mfull = jnp.concatenate(
        [jnp.broadcast_to(m, (8, 16)), jnp.full((8, 112), -jnp.inf, _F32)],
        axis=1)

    @pl.when(i == 0)
    def _():
        scr[...] = jnp.full((8, 128), -jnp.inf, _F32)

    scr[...] = jnp.maximum(scr[...], mfull)

    @pl.when(i == _NBLK - 1)
    def _():
        g8 = scr[0:1, 0:8] + scr[0:1, 8:16]
        g_ref[0:1, 0:8] = g8
        g_ref[0:1, 8:16] = g8


def _build_tca(interpret=False):
    return pl.pallas_call(
        _tca_body,
        grid=(_NBLK,),
        in_specs=[
            pl.BlockSpec((_BLK, 128), lambda i: (i, 0)),
            pl.BlockSpec((128, 128), lambda i: (0, 0)),
            pl.BlockSpec((128, 16), lambda i: (0, 0)),
        ],
        out_specs=[
            pl.BlockSpec((_BLK, _ROW1), lambda i: (i, 0)),
            pl.BlockSpec((_BLK, 16), lambda i: (i, 0)),
            pl.BlockSpec((1, 16), lambda i: (0, 0)),
        ],
        out_shape=[
            jax.ShapeDtypeStruct((_NP, _ROW1), _F32),
            jax.ShapeDtypeStruct((_NP, 16), _F32),
            jax.ShapeDtypeStruct((1, 16), _F32),
        ],
        scratch_shapes=[pltpu.VMEM((8, 128), _F32)],
        interpret=interpret,
    )


def _tcb_body(a_ref, r_ref, b1_ref, w2_ref, a2_ref, p2_ref,
              pk_ref, ad_ref, g_ref, scr):
    i = pl.program_id(0)
    acc = a_ref[0] + a_ref[1]
    num = acc[:, 0:128]
    den8 = acc[:, 128:136]
    den = jnp.dot(den8, r_ref[...], preferred_element_type=_F32)
    out1 = num / (den + 1e-16) + b1_ref[...]
    helu = jnp.where(out1 > 0, out1, jnp.exp(jnp.minimum(out1, 0.0)) - 1.0)
    h2 = jnp.dot(helu, w2_ref[...], preferred_element_type=_F32)
    a2 = jnp.dot(h2, a2_ref[...], preferred_element_type=_F32)
    pk_ref[:, 0:16] = h2
    pk_ref[:, 16:32] = a2
    ad_ref[...] = jnp.dot(a2, p2_ref[...], preferred_element_type=_F32)
    m = jnp.max(a2, axis=0, keepdims=True)
    mfull = jnp.concatenate(
        [jnp.broadcast_to(m, (8, 16)), jnp.full((8, 112), -jnp.inf, _F32)],
        axis=1)

    @pl.when(i == 0)
    def _():
        scr[...] = jnp.full((8, 128), -jnp.inf, _F32)

    scr[...] = jnp.maximum(scr[...], mfull)

    @pl.when(i == _NBLK - 1)
    def _():
        g2 = scr[0:1, 0:1] + scr[0:1, 1:2]
        g_ref[...] = jnp.broadcast_to(g2, (1, 16))


def _build_tcb(interpret=False):
    return pl.pallas_call(
        _tcb_body,
        grid=(_NBLK,),
        in_specs=[
            pl.BlockSpec((2, _BLK, _ROW1), lambda i: (0, i, 0)),
            pl.BlockSpec((8, 128), lambda i: (0, 0)),
            pl.BlockSpec((1, 128), lambda i: (0, 0)),
            pl.BlockSpec((128, 16), lambda i: (0, 0)),
            pl.BlockSpec((16, 16), lambda i: (0, 0)),
            pl.BlockSpec((16, 16), lambda i: (0, 0)),
        ],
        out_specs=[
            pl.BlockSpec((_BLK, _ROW2), lambda i: (i, 0)),
            pl.BlockSpec((_BLK, 16), lambda i: (i, 0)),
            pl.BlockSpec((1, 16), lambda i: (0, 0)),
        ],
        out_shape=[
            jax.ShapeDtypeStruct((_NP, _ROW2), _F32),
            jax.ShapeDtypeStruct((_NP, 16), _F32),
            jax.ShapeDtypeStruct((1, 16), _F32),
        ],
        scratch_shapes=[pltpu.VMEM((8, 128), _F32)],
        interpret=interpret,
    )


def _tcc_body(a_ref, b2_ref, o_ref):
    acc = a_ref[0] + a_ref[1]
    num = acc[:, 0:16]
    den = acc[:, 16:17]
    o = num / (den + 1e-16) + b2_ref[...]
    m = jnp.max(o, axis=1, keepdims=True)
    sh = o - m
    lse = jnp.log(jnp.sum(jnp.exp(sh), axis=1, keepdims=True))
    o_ref[...] = sh - lse


def _build_tcc(interpret=False):
    return pl.pallas_call(
        _tcc_body,
        grid=(_NBLK,),
        in_specs=[
            pl.BlockSpec((2, _BLK, _ROW2), lambda i: (0, i, 0)),
            pl.BlockSpec((1, 16), lambda i: (0, 0)),
        ],
        out_specs=pl.BlockSpec((_BLK, 16), lambda i: (i, 0)),
        out_shape=jax.ShapeDtypeStruct((_NP, 16), _F32),
        interpret=interpret,
    )


def _build_sc_edge(nheads, nc, ns, ep, B, cps, interpret=False):
    row = nheads * 16 + 16
    att_off = nheads * 16
    SB = B * cps
    nsb = ep // SB
    rows_per_sub = _NP // ns
    nvi = B // 16
    mesh = plsc.VectorSubcoreMesh(
        core_axis_name="c", subcore_axis_name="s",
        num_cores=nc, num_subcores=ns)

    @functools.partial(
        pl.kernel,
        out_type=jax.ShapeDtypeStruct((nc, _NP, row), _F32),
        mesh=mesh,
        scratch_types=[
            pltpu.VMEM((SB,), jnp.int32),
            pltpu.VMEM((SB,), jnp.int32),
            pltpu.VMEM((SB,), jnp.int32),
            pltpu.VMEM((SB,), jnp.int32),
            pltpu.VMEM((B,), jnp.int32),
            pltpu.VMEM((B,), jnp.int32),
            pltpu.VMEM((B,), jnp.int32),
            pltpu.VMEM((B,), jnp.int32),
            pltpu.VMEM((B,), jnp.int32),
            pltpu.VMEM((B,), jnp.int32),
            pltpu.VMEM((B, row), _F32),
            pltpu.VMEM((B, 16), _F32),
            pltpu.VMEM((B, row), _F32),
            pltpu.VMEM((B, 16), _F32),
            pltpu.VMEM((B, row), _F32),
            pltpu.VMEM((B, 16), _F32),
            pltpu.VMEM((16,), _F32),
            pltpu.VMEM_SHARED((_NP, row), _F32),
            pltpu.SemaphoreType.DMA,
            pltpu.SemaphoreType.DMA,
            pltpu.SemaphoreType.DMA,
            pltpu.SemaphoreType.DMA,
            pltpu.SemaphoreType.DMA,
            pltpu.SemaphoreType.DMA,
            pltpu.SemaphoreType.DMA,
            pltpu.SemaphoreType.DMA,
        ],
        compiler_params=pltpu.CompilerParams(
            use_tc_tiling_on_sc=False, needs_layout_passes=False),
        interpret=interpret,
    )
    def sc_edge(pack_hbm, adt_hbm, ei_hbm, gmax_hbm, zrow_hbm,
                out_hbm, ix0, ix1, dx0, dx1, ids0, ids1, ids2, idd0, idd1,
                idd2, g0, a0, g1, a1, g2, a2, gmaxv, acc, semi0, semi1,
                semg0, semg1, semg2, sems0, sems1, sems2):
        c = lax.axis_index("c")
        s = lax.axis_index("s")
        w = s * nc + c
        ebase = w * ep
        pltpu.sync_copy(zrow_hbm, acc.at[pl.ds(s * rows_per_sub, rows_per_sub)])
        pltpu.sync_copy(gmax_hbm, gmaxv)
        plsc.subcore_barrier()
        gv = gmaxv[...]
        lane = lax.iota(jnp.int32, 16)
        ibufs = ((ix0, dx0, semi0), (ix1, dx1, semi1))
        slots = ((ids0, idd0, g0, a0, semg0, sems0),
                 (ids1, idd1, g1, a1, semg1, sems1),
                 (ids2, idd2, g2, a2, semg2, sems2))

        def idx_issue(sidx, ib):
            ix, dx, sem = ibufs[ib]
            base = ebase + sidx * SB
            pltpu.async_copy(ei_hbm.at[0, pl.ds(base, SB)], ix, sem)
            pltpu.async_copy(ei_hbm.at[1, pl.ds(base, SB)], dx, sem)

        def idx_wait(sidx, ib):
            ix, dx, sem = ibufs[ib]
            base = ebase + sidx * SB
            pltpu.make_async_copy(ei_hbm.at[0, pl.ds(base, SB)], ix, sem).wait()
            pltpu.make_async_copy(ei_hbm.at[1, pl.ds(base, SB)], dx, sem).wait()

        def gather_issue(ib, j, sl):
            ix, dx, _ = ibufs[ib]
            ids, idd, gb, ab, sem, _ = slots[sl]
            for q in range(nvi):
                ids[pl.ds(q * 16, 16)] = ix[pl.ds(j * B + q * 16, 16)]
                idd[pl.ds(q * 16, 16)] = dx[pl.ds(j * B + q * 16, 16)]
            pltpu.async_copy(pack_hbm.at[ids], gb, sem)
            pltpu.async_copy(adt_hbm.at[idd], ab, sem)

        def gather_wait(sl):
            ids, idd, gb, ab, sem, _ = slots[sl]
            pltpu.make_async_copy(pack_hbm.at[ids], gb, sem).wait()
            pltpu.make_async_copy(adt_hbm.at[idd], ab, sem).wait()

        def scatter_wait(sl):
            ids, idd, gb, ab, _, sem = slots[sl]
            pltpu.make_async_copy(gb, acc.at[idd], sem).wait()

        c_att = jnp.full((16,), att_off, jnp.int32)
        c_zero = jnp.zeros((16,), jnp.int32)

        def compute_chunk(sl):
            ids, idd, gb, ab, _, sem = slots[sl]
            if nheads == 1:
                def grp(p, carry):
                    e0 = p * 16
                    rws = e0 + lane
                    va = plsc.load_gather(gb, [rws, c_att])
                    vb = plsc.load_gather(ab, [rws, c_zero])
                    al = va + vb
                    al = jnp.where(al >= 0.0, al, al * 0.2)
                    ea = jnp.exp(al - gv)
                    plsc.store_scatter(gb, [rws, c_att], ea)
                    for t in range(16):
                        e = e0 + t
                        gb[e, pl.ds(0, 16)] = gb[e, pl.ds(0, 16)] * ea[t]
                    return carry

                lax.fori_loop(0, B // 16, grp, 0, unroll=2)
            else:
                rofs = lax.shift_right_logical(lane, 3)
                hofs = lane & 7

                def pair(p, carry):
                    e0 = p * 2
                    rws = e0 + rofs
                    cls = att_off + hofs
                    va = plsc.load_gather(gb, [rws, cls])
                    vb = plsc.load_gather(ab, [rws, hofs])
                    al = va + vb
                    al = jnp.where(al >= 0.0, al, al * 0.2)
                    ea = jnp.exp(al - gv)
                    plsc.store_scatter(gb, [rws, cls], ea)
                    for t in range(2):
                        e = e0 + t
                        for j in range(nheads):
                            sj = ea[8 * t + j]
                            gb[e, pl.ds(16 * j, 16)] = (
                                gb[e, pl.ds(16 * j, 16)] * sj)
                    return carry

                lax.fori_loop(0, B // 2, pair, 0, unroll=2)
            pltpu.async_copy(gb, acc.at[idd], sem, add=True)

        idx_issue(0, 0)
        idx_wait(0, 0)
        gather_issue(0, 0, 0)

        def outer(s2, carry):
            for sb in range(2):
                sidx = s2 * 2 + sb
                nsidx = lax.rem(sidx + 1, nsb)
                idx_issue(nsidx, 1 - sb)
                for j in range(cps):
                    sl = j
                    nsl = (j + 1) % 3
                    if sb == 0 and j < 2:
                        @pl.when(s2 > 0)
                        def _():
                            scatter_wait(nsl)
                    else:
                        scatter_wait(nsl)
                    if j == cps - 1:
                        idx_wait(nsidx, 1 - sb)
                        gather_issue(1 - sb, 0, nsl)
                    else:
                        gather_issue(sb, j + 1, nsl)
                    gather_wait(sl)
                    compute_chunk(sl)
            return carry

        lax.fori_loop(0, nsb // 2, outer, 0)
        gather_wait(0)
        scatter_wait(1)
        scatter_wait(2)
        plsc.subcore_barrier()
        pltpu.sync_copy(
            acc.at[pl.ds(s * rows_per_sub, rows_per_sub)],
            out_hbm.at[c, pl.ds(s * rows_per_sub, rows_per_sub)])

    return sc_edge


def _sc_geometry():
    try:
        info = plsc.get_sparse_core_info()
        nc, ns = info.num_cores, info.num_subcores
    except Exception:
        nc, ns = 2, 16
    return nc, ns


def kernel(x, edge_index, W1, att_src1, att_dst1, b1, W2, att_src2,
           att_dst2, b2):
    nc, ns = _sc_geometry()
    nw = nc * ns
    etot = _E + _N
    b1c, b2c, cps = 80, 96, 3
    ep1 = math.ceil(etot / (nw * 2 * b1c * cps)) * 2 * b1c * cps
    ep2 = math.ceil(etot / (nw * 2 * b2c * cps)) * 2 * b2c * cps
    npad = max(nw * ep1, nw * ep2) - etot
    loop = jnp.arange(_N, dtype=jnp.int32)
    padidx = _N + (jnp.arange(npad, dtype=jnp.int32) % 16)
    ei = jnp.concatenate(
        [edge_index.astype(jnp.int32),
         jnp.broadcast_to(loop, (2, _N)),
         jnp.broadcast_to(padidx, (2, npad))], axis=1)

    xp = jnp.pad(x, ((0, _NP - _N), (0, 0)))
    mask8 = (jnp.arange(128)[:, None] // 16 == jnp.arange(8)[None, :])
    AA = jnp.concatenate([
        jnp.where(mask8, att_src1.reshape(-1)[:, None], 0.0),
        jnp.where(mask8, att_dst1.reshape(-1)[:, None], 0.0)], axis=1)

    tca = _build_tca()
    pack1, adt1, gmacc = tca(xp, W1, AA)
    gmax16 = gmacc.reshape(16)

    sc1 = _build_sc_edge(_HEADS, nc, ns, ep1, b1c, cps)
    z1 = jnp.zeros((_NP // ns, _ROW1), _F32)
    acc1 = sc1(pack1, adt1, ei, gmax16, z1)

    R = (jnp.arange(128)[None, :] // 16 == jnp.arange(8)[:, None]).astype(_F32)
    A2 = jnp.zeros((16, 16), _F32)
    A2 = A2.at[:, 0].set(att_src2[0]).at[:, 1].set(att_dst2[0])
    P2 = jnp.zeros((16, 16), _F32).at[1, 0].set(1.0)
    tcb = _build_tcb()
    pack2, adt2, gm2acc = tcb(acc1, R, b1.reshape(1, 128), W2, A2, P2)
    gmax2 = gm2acc.reshape(16)

    sc2 = _build_sc_edge(1, nc, ns, ep2, b2c, cps)
    z2 = jnp.zeros((_NP // ns, _ROW2), _F32)
    acc2 = sc2(pack2, adt2, ei, gmax2, z2)

    tcc = _build_tcc()
    o = tcc(acc2, b2.reshape(1, 16))
    return o[:_N]


# speedup vs baseline: 143.6314x; 1.0172x over previous
import functools
import math

import jax
import jax.numpy as jnp
from jax import lax
from jax.experimental import pallas as pl
from jax.experimental.pallas import tpu as pltpu
from jax.experimental.pallas import tpu_sc as plsc

_N = 10000
_E = 320000
_IN = 128
_HID = 16
_HEADS = 8
_OUT = 16
_BLK = 2528
_NBLK = 4
_NP = _BLK * _NBLK
_ROW1 = _HEADS * 16 + 16
_ROW2 = 1 * 16 + 16
_F32 = jnp.float32


def _tca_body(x_ref, w_ref, aa_ref, p_ref, ad_ref, g_ref, scr):
    i = pl.program_id(0)
    h = jnp.dot(x_ref[...], w_ref[...], preferred_element_type=_F32)
    aab = jnp.dot(h, aa_ref[...], preferred_element_type=_F32)
    p_ref[:, 0:128] = h
    p_ref[:, 128:144] = aab
    ad_ref[:, 0:8] = aab[:, 8:16]
    ad_ref[:, 8:16] = jnp.zeros((_BLK, 8), _F32)
    m = jnp.max(aab, axis=0, keepdims=True)
    mfull = jnp.concatenate(
        [jnp.broadcast_to(m, (8, 16)), jnp.full((8, 112), -jnp.inf, _F32)],
        axis=1)

    @pl.when(i == 0)
    def _():
        scr[...] = jnp.full((8, 128), -jnp.inf, _F32)

    scr[...] = jnp.maximum(scr[...], mfull)

    @pl.when(i == _NBLK - 1)
    def _():
        g8 = scr[0:1, 0:8] + scr[0:1, 8:16]
        g_ref[0:1, 0:8] = g8
        g_ref[0:1, 8:16] = g8


def _build_tca(interpret=False):
    return pl.pallas_call(
        _tca_body,
        grid=(_NBLK,),
        in_specs=[
            pl.BlockSpec((_BLK, 128), lambda i: (i, 0)),
            pl.BlockSpec((128, 128), lambda i: (0, 0)),
            pl.BlockSpec((128, 16), lambda i: (0, 0)),
        ],
        out_specs=[
            pl.BlockSpec((_BLK, _ROW1), lambda i: (i, 0)),
            pl.BlockSpec((_BLK, 16), lambda i: (i, 0)),
            pl.BlockSpec((1, 16), lambda i: (0, 0)),
        ],
        out_shape=[
            jax.ShapeDtypeStruct((_NP, _ROW1), _F32),
            jax.ShapeDtypeStruct((_NP, 16), _F32),
            jax.ShapeDtypeStruct((1, 16), _F32),
        ],
        scratch_shapes=[pltpu.VMEM((8, 128), _F32)],
        interpret=interpret,
    )


def _tcb_body(a_ref, r_ref, b1_ref, w2_ref, a2_ref, p2_ref,
              pk_ref, ad_ref, g_ref, scr):
    i = pl.program_id(0)
    acc = a_ref[0] + a_ref[1]
    num = acc[:, 0:128]
    den8 = acc[:, 128:136]
    den = jnp.dot(den8, r_ref[...], preferred_element_type=_F32)
    out1 = num / (den + 1e-16) + b1_ref[...]
    helu = jnp.where(out1 > 0, out1, jnp.exp(jnp.minimum(out1, 0.0)) - 1.0)
    h2 = jnp.dot(helu, w2_ref[...], preferred_element_type=_F32)
    a2 = jnp.dot(h2, a2_ref[...], preferred_element_type=_F32)
    pk_ref[:, 0:16] = h2
    pk_ref[:, 16:32] = a2
    ad_ref[...] = jnp.dot(a2, p2_ref[...], preferred_element_type=_F32)
    m = jnp.max(a2, axis=0, keepdims=True)
    mfull = jnp.concatenate(
        [jnp.broadcast_to(m, (8, 16)), jnp.full((8, 112), -jnp.inf, _F32)],
        axis=1)

    @pl.when(i == 0)
    def _():
        scr[...] = jnp.full((8, 128), -jnp.inf, _F32)

    scr[...] = jnp.maximum(scr[...], mfull)

    @pl.when(i == _NBLK - 1)
    def _():
        g2 = scr[0:1, 0:1] + scr[0:1, 1:2]
        g_ref[...] = jnp.broadcast_to(g2, (1, 16))


def _build_tcb(interpret=False):
    return pl.pallas_call(
        _tcb_body,
        grid=(_NBLK,),
        in_specs=[
            pl.BlockSpec((2, _BLK, _ROW1), lambda i: (0, i, 0)),
            pl.BlockSpec((8, 128), lambda i: (0, 0)),
            pl.BlockSpec((1, 128), lambda i: (0, 0)),
            pl.BlockSpec((128, 16), lambda i: (0, 0)),
            pl.BlockSpec((16, 16), lambda i: (0, 0)),
            pl.BlockSpec((16, 16), lambda i: (0, 0)),
        ],
        out_specs=[
            pl.BlockSpec((_BLK, _ROW2), lambda i: (i, 0)),
            pl.BlockSpec((_BLK, 16), lambda i: (i, 0)),
            pl.BlockSpec((1, 16), lambda i: (0, 0)),
        ],
        out_shape=[
            jax.ShapeDtypeStruct((_NP, _ROW2), _F32),
            jax.ShapeDtypeStruct((_NP, 16), _F32),
            jax.ShapeDtypeStruct((1, 16), _F32),
        ],
        scratch_shapes=[pltpu.VMEM((8, 128), _F32)],
        interpret=interpret,
    )


def _tcc_body(a_ref, b2_ref, o_ref):
    acc = a_ref[0] + a_ref[1]
    num = acc[:, 0:16]
    den = acc[:, 16:17]
    o = num / (den + 1e-16) + b2_ref[...]
    m = jnp.max(o, axis=1, keepdims=True)
    sh = o - m
    lse = jnp.log(jnp.sum(jnp.exp(sh), axis=1, keepdims=True))
    o_ref[...] = sh - lse


def _build_tcc(interpret=False):
    return pl.pallas_call(
        _tcc_body,
        grid=(_NBLK,),
        in_specs=[
            pl.BlockSpec((2, _BLK, _ROW2), lambda i: (0, i, 0)),
            pl.BlockSpec((1, 16), lambda i: (0, 0)),
        ],
        out_specs=pl.BlockSpec((_BLK, 16), lambda i: (i, 0)),
        out_shape=jax.ShapeDtypeStruct((_NP, 16), _F32),
        interpret=interpret,
    )


def _build_sc_edge(nheads, nc, ns, ep, B, cps, interpret=False):
    row = nheads * 16 + 16
    att_off = nheads * 16
    SB = B * cps
    nsb = ep // SB
    rows_per_sub = _NP // ns
    nvi = B // 16
    mesh = plsc.VectorSubcoreMesh(
        core_axis_name="c", subcore_axis_name="s",
        num_cores=nc, num_subcores=ns)

    @functools.partial(
        pl.kernel,
        out_type=jax.ShapeDtypeStruct((nc, _NP, row), _F32),
        mesh=mesh,
        scratch_types=[
            pltpu.VMEM((SB,), jnp.int32),
            pltpu.VMEM((SB,), jnp.int32),
            pltpu.VMEM((SB,), jnp.int32),
            pltpu.VMEM((SB,), jnp.int32),
            pltpu.VMEM((B,), jnp.int32),
            pltpu.VMEM((B,), jnp.int32),
            pltpu.VMEM((B,), jnp.int32),
            pltpu.VMEM((B,), jnp.int32),
            pltpu.VMEM((B,), jnp.int32),
            pltpu.VMEM((B,), jnp.int32),
            pltpu.VMEM((B, row), _F32),
            pltpu.VMEM((B, 16), _F32),
            pltpu.VMEM((B, row), _F32),
            pltpu.VMEM((B, 16), _F32),
            pltpu.VMEM((B, row), _F32),
            pltpu.VMEM((B, 16), _F32),
            pltpu.VMEM((16,), _F32),
            pltpu.VMEM_SHARED((_NP, row), _F32),
            pltpu.SemaphoreType.DMA,
            pltpu.SemaphoreType.DMA,
            pltpu.SemaphoreType.DMA,
            pltpu.SemaphoreType.DMA,
            pltpu.SemaphoreType.DMA,
            pltpu.SemaphoreType.DMA,
            pltpu.SemaphoreType.DMA,
            pltpu.SemaphoreType.DMA,
        ],
        compiler_params=pltpu.CompilerParams(
            use_tc_tiling_on_sc=False, needs_layout_passes=False),
        interpret=interpret,
    )
    def sc_edge(pack_hbm, adt_hbm, ei_hbm, gmax_hbm, zrow_hbm,
                out_hbm, ix0, ix1, dx0, dx1, ids0, ids1, ids2, idd0, idd1,
                idd2, g0, a0, g1, a1, g2, a2, gmaxv, acc, semi0, semi1,
                semg0, semg1, semg2, sems0, sems1, sems2):
        c = lax.axis_index("c")
        s = lax.axis_index("s")
        w = s * nc + c
        ebase = w * ep
        pltpu.sync_copy(zrow_hbm, acc.at[pl.ds(s * rows_per_sub, rows_per_sub)])
        pltpu.sync_copy(gmax_hbm, gmaxv)
        plsc.subcore_barrier()
        gv = gmaxv[...]
        lane = lax.iota(jnp.int32, 16)
        ibufs = ((ix0, dx0, semi0), (ix1, dx1, semi1))
        slots = ((ids0, idd0, g0, a0, semg0, sems0),
                 (ids1, idd1, g1, a1, semg1, sems1),
                 (ids2, idd2, g2, a2, semg2, sems2))

        def idx_issue(sidx, ib):
            ix, dx, sem = ibufs[ib]
            base = ebase + sidx * SB
            pltpu.async_copy(ei_hbm.at[0, pl.ds(base, SB)], ix, sem)
            pltpu.async_copy(ei_hbm.at[1, pl.ds(base, SB)], dx, sem)

        def idx_wait(sidx, ib):
            ix, dx, sem = ibufs[ib]
            base = ebase + sidx * SB
            pltpu.make_async_copy(ei_hbm.at[0, pl.ds(base, SB)], ix, sem).wait()
            pltpu.make_async_copy(ei_hbm.at[1, pl.ds(base, SB)], dx, sem).wait()

        def gather_issue(ib, j, sl):
            ix, dx, _ = ibufs[ib]
            ids, idd, gb, ab, sem, _ = slots[sl]
            for q in range(nvi):
                ids[pl.ds(q * 16, 16)] = ix[pl.ds(j * B + q * 16, 16)]
                idd[pl.ds(q * 16, 16)] = dx[pl.ds(j * B + q * 16, 16)]
            pltpu.async_copy(pack_hbm.at[ids], gb, sem)
            pltpu.async_copy(adt_hbm.at[idd], ab, sem)

        def gather_wait(sl):
            ids, idd, gb, ab, sem, _ = slots[sl]
            pltpu.make_async_copy(pack_hbm.at[ids], gb, sem).wait()
            pltpu.make_async_copy(adt_hbm.at[idd], ab, sem).wait()

        def scatter_wait(sl):
            ids, idd, gb, ab, _, sem = slots[sl]
            pltpu.make_async_copy(gb, acc.at[idd], sem).wait()

        c_att = jnp.full((16,), att_off, jnp.int32)
        c_zero = jnp.zeros((16,), jnp.int32)

        def compute_chunk(sl):
            ids, idd, gb, ab, _, sem = slots[sl]
            if nheads == 1:
                def grp(p, carry):
                    e0 = p * 16
                    rws = e0 + lane
                    va = plsc.load_gather(gb, [rws, c_att])
                    vb = plsc.load_gather(ab, [rws, c_zero])
                    al = va + vb
                    al = jnp.where(al >= 0.0, al, al * 0.2)
                    ea = jnp.exp(al - gv)
                    plsc.store_scatter(gb, [rws, c_att], ea)
                    for t in range(16):
                        e = e0 + t
                        gb[e, pl.ds(0, 16)] = gb[e, pl.ds(0, 16)] * ea[t]
                    return carry

                lax.fori_loop(0, B // 16, grp, 0, unroll=2)
            else:
                rofs = lax.shift_right_logical(lane, 3)
                hofs = lane & 7

                def pair(p, carry):
                    e0 = p * 2
                    rws = e0 + rofs
                    cls = att_off + hofs
                    va = plsc.load_gather(gb, [rws, cls])
                    vb = plsc.load_gather(ab, [rws, hofs])
                    al = va + vb
                    al = jnp.where(al >= 0.0, al, al * 0.2)
                    ea = jnp.exp(al - gv)
                    plsc.store_scatter(gb, [rws, cls], ea)
                    for t in range(2):
                        e = e0 + t
                        for j in range(nheads):
                            sj = ea[8 * t + j]
                            gb[e, pl.ds(16 * j, 16)] = (
                                gb[e, pl.ds(16 * j, 16)] * sj)
                    return carry

                lax.fori_loop(0, B // 2, pair, 0, unroll=2)
            pltpu.async_copy(gb, acc.at[idd], sem, add=True)

        idx_issue(0, 0)
        idx_wait(0, 0)
        gather_issue(0, 0, 0)

        def outer(s2, carry):
            for sb in range(2):
                sidx = s2 * 2 + sb
                nsidx = lax.rem(sidx + 1, nsb)
                idx_issue(nsidx, 1 - sb)
                for j in range(cps):
                    sl = j
                    nsl = (j + 1) % 3
                    if sb == 0 and j < 2:
                        @pl.when(s2 > 0)
                        def _():
                            scatter_wait(nsl)
                    else:
                        scatter_wait(nsl)
                    if j == cps - 1:
                        idx_wait(nsidx, 1 - sb)
                        gather_issue(1 - sb, 0, nsl)
                    else:
                        gather_issue(sb, j + 1, nsl)
                    gather_wait(sl)
                    compute_chunk(sl)
            return carry

        lax.fori_loop(0, nsb // 2, outer, 0)
        gather_wait(0)
        scatter_wait(1)
        scatter_wait(2)
        plsc.subcore_barrier()
        pltpu.sync_copy(
            acc.at[pl.ds(s * rows_per_sub, rows_per_sub)],
            out_hbm.at[c, pl.ds(s * rows_per_sub, rows_per_sub)])

    return sc_edge


def _sc_geometry():
    try:
        info = plsc.get_sparse_core_info()
        nc, ns = info.num_cores, info.num_subcores
    except Exception:
        nc, ns = 2, 16
    return nc, ns


def kernel(x, edge_index, W1, att_src1, att_dst1, b1, W2, att_src2,
           att_dst2, b2):
    nc, ns = _sc_geometry()
    nw = nc * ns
    etot = _E + _N
    b1c, b2c, cps = 80, 96, 3
    ep1 = math.ceil(etot / (nw * 2 * b1c * cps)) * 2 * b1c * cps
    ep2 = math.ceil(etot / (nw * 2 * b2c * cps)) * 2 * b2c * cps
    npad = max(nw * ep1, nw * ep2) - etot
    loop = jnp.arange(_N, dtype=jnp.int32)
    padidx = _N + (jnp.arange(npad, dtype=jnp.int32) % 16)
    ei = jnp.concatenate(
        [edge_index.astype(jnp.int32),
         jnp.broadcast_to(loop, (2, _N)),
         jnp.broadcast_to(padidx, (2, npad))], axis=1)

    xp = jnp.pad(x, ((0, _NP - _N), (0, 0)))
    mask8 = (jnp.arange(128)[:, None] // 16 == jnp.arange(8)[None, :])
    AA = jnp.concatenate([
        jnp.where(mask8, att_src1.reshape(-1)[:, None], 0.0),
        jnp.where(mask8, att_dst1.reshape(-1)[:, None], 0.0)], axis=1)

    tca = _build_tca()
    pack1, adt1, gmacc = tca(xp, W1, AA)
    gmax16 = gmacc.reshape(16)

    sc1 = _build_sc_edge(_HEADS, nc, ns, ep1, b1c, cps)
    z1 = jnp.zeros((_NP // ns, _ROW1), _F32)
    acc1 = sc1(pack1, adt1, ei, gmax16, z1)

    R = (jnp.arange(128)[None, :] // 16 == jnp.arange(8)[:, None]).astype(_F32)
    A2 = jnp.zeros((16, 16), _F32)
    A2 = A2.at[:, 0].set(att_src2[0]).at[:, 1].set(att_dst2[0])
    P2 = jnp.zeros((16, 16), _F32).at[1, 0].set(1.0)
    tcb = _build_tcb()
    pack2, adt2, gm2acc = tcb(acc1, R, b1.reshape(1, 128), W2, A2, P2)
    gmax2 = gm2acc.reshape(16)

    sc2 = _build_sc_edge(1, nc, ns, ep2, b2c, cps)
    z2 = jnp.zeros((_NP // ns, _ROW2), _F32)
    acc2 = sc2(pack2, adt2, ei, gmax2, z2)

    tcc = _build_tcc()
    o = tcc(acc2, b2.reshape(1, 16))
    return o[:_N]
